# baseline (speedup 1.0000x reference)
import functools

import jax
import jax.numpy as jnp
from jax import lax
from jax.experimental import pallas as pl
from jax.experimental.pallas import tpu as pltpu
from jax.experimental.pallas import tpu_sc as plsc

NC, NS, L = 2, 16, 16
NW = NC * NS
BT = 128


def _shift_down(a, sh):
    z = jnp.zeros((sh,) + a.shape[1:], a.dtype)
    return jnp.concatenate([z, a[:-sh]], axis=0)


def _shift_right(a, sh):
    z = jnp.zeros(a.shape[:1] + (sh,) + a.shape[2:], a.dtype)
    return jnp.concatenate([z, a[:, :-sh]], axis=1)


def _router_body(x_ref, rw_ref, meta_i_ref, meta_f_ref, *, T, E, NRT, NTE):
    x = x_ref[...]
    logits = lax.dot_general(x, rw_ref[...], (((1,), (1,)), ((), ())),
                             preferred_element_type=jnp.float32)
    C = logits.shape[1]
    cols = lax.broadcasted_iota(jnp.int32, (T, C), 1)
    is_e = cols < E
    el = jnp.where(is_e, logits, -1e30)
    m = jnp.max(el, axis=1, keepdims=True)
    p = jnp.where(is_e, jnp.exp(el - m), 0.0)
    p = p / jnp.sum(p, axis=1, keepdims=True)

    p1 = jnp.max(p, axis=1, keepdims=True)
    a1 = jnp.min(jnp.where(p == p1, cols, C), axis=1, keepdims=True)
    p_wo = jnp.where(cols == a1, -1.0, p)
    p2 = jnp.max(p_wo, axis=1, keepdims=True)
    a2 = jnp.min(jnp.where(p_wo == p2, cols, C), axis=1, keepdims=True)
    wsum = p1 + p2
    w1, w2 = p1 / wsum, p2 / wsum
    g = 1.0 / (1.0 + jnp.exp(-logits[:, E:E + 1]))

    m0 = (cols == a1)
    m1 = (cols == a2)
    mm = (m0 | m1).astype(jnp.int32)
    c = mm
    sh = 1
    while sh < T:
        c = c + _shift_down(c, sh)
        sh *= 2
    counts = c[T - 1:T, :]
    excl = c - mm
    rank0 = jnp.sum(jnp.where(m0, excl, 0), axis=1, keepdims=True)
    rank1 = jnp.sum(jnp.where(m1, excl, 0), axis=1, keepdims=True)

    nt = (counts + (BT - 1)) // BT
    ts = nt
    sh = 1
    while sh < C:
        ts = ts + _shift_right(ts, sh)
        sh *= 2
    tile_start = ts - nt
    off = tile_start * BT
    pos0 = jnp.sum(jnp.where(m0, off, 0), axis=1, keepdims=True) + rank0
    pos1 = jnp.sum(jnp.where(m1, off, 0), axis=1, keepdims=True) + rank1

    ti = lax.broadcasted_iota(jnp.int32, (NTE, C), 0)
    tcols = lax.broadcasted_iota(jnp.int32, (NTE, C), 1)
    ts_b = jnp.broadcast_to(tile_start, (NTE, C))
    nt_b = jnp.broadcast_to(nt, (NTE, C))
    ind = ((ti >= ts_b) & (ti < ts_b + nt_b) & (tcols < E)).astype(jnp.int32)
    any_ind = jnp.sum(ind, axis=1, keepdims=True)
    te = jnp.sum(ind * tcols, axis=1, keepdims=True) - (1 - any_ind)
    te = jnp.where(ti[:, :1] >= NRT, E, te)

    meta_i_ref[:, 0:1] = pos0
    meta_i_ref[:, 1:2] = pos1
    meta_i_ref[0:NTE, 2:3] = te
    meta_f_ref[:, 0:L] = jnp.broadcast_to(w1, (T, L))
    meta_f_ref[:, L:2 * L] = jnp.broadcast_to(w2, (T, L))
    meta_f_ref[:, 2 * L:3 * L] = jnp.broadcast_to(g, (T, L))


def _dispatch_body(x_hbm, pos0_hbm, pos1_hbm, xs_hbm, buf, idx0, idx1, sem0, sem1,
                   *, chunk):
    wid = lax.axis_index("s") * NC + lax.axis_index("c")
    base = pl.multiple_of(wid * chunk, 8)
    pltpu.sync_copy(x_hbm.at[pl.ds(base, chunk)], buf)
    pltpu.sync_copy(pos0_hbm.at[pl.ds(base, chunk)], idx0)
    pltpu.sync_copy(pos1_hbm.at[pl.ds(base, chunk)], idx1)
    c0 = pltpu.async_copy(buf, xs_hbm.at[idx0], sem0)
    c1 = pltpu.async_copy(buf, xs_hbm.at[idx1], sem1)
    c0.wait()
    c1.wait()


def _ffn_body(te_ref, xs_ref, x_ref, wgu_ref, wd_ref, o_ref, *, NRT, I):
    i = pl.program_id(0)
    tev = te_ref[i]
    rows = jnp.where(i < NRT, xs_ref[...], x_ref[...])

    @pl.when(tev >= 0)
    def _():
        gu = lax.dot_general(rows, wgu_ref[0], (((1,), (1,)), ((), ())),
                             preferred_element_type=jnp.float32)
        gt = gu[:, :I]
        up = gu[:, I:]
        act = gt * (1.0 / (1.0 + jnp.exp(-gt))) * up
        o_ref[...] = lax.dot_general(act.astype(wd_ref.dtype), wd_ref[0],
                                     (((1,), (1,)), ((), ())),
                                     preferred_element_type=jnp.float32)


def _combine_body(oall_hbm, pos0_hbm, pos1_hbm, mf_hbm, out_hbm,
                  a_buf, b_buf, s_buf, w_buf, idx0, idx1, sem0, sem1,
                  *, rows, n_rounds, shared_base, H):
    wid = lax.axis_index("s") * NC + lax.axis_index("c")
    nch = H // L

    def round_body(rnd, _):
        base = pl.multiple_of(wid * (rows * n_rounds) + rnd * rows, 8)
        pltpu.sync_copy(pos0_hbm.at[pl.ds(base, rows)], idx0)
        pltpu.sync_copy(pos1_hbm.at[pl.ds(base, rows)], idx1)
        c0 = pltpu.async_copy(oall_hbm.at[idx0], a_buf, sem0)
        c1 = pltpu.async_copy(oall_hbm.at[idx1], b_buf, sem1)
        pltpu.sync_copy(oall_hbm.at[pl.ds(shared_base + base, rows)], s_buf)
        pltpu.sync_copy(mf_hbm.at[pl.ds(base, rows)], w_buf)
        c0.wait()
        c1.wait()

        @plsc.parallel_loop(0, rows)
        def row_body(r):
            wv0 = w_buf[r, pl.ds(0, L)]
            wv1 = w_buf[r, pl.ds(L, L)]
            wvg = w_buf[r, pl.ds(2 * L, L)]

            @plsc.parallel_loop(0, nch, unroll=8)
            def chunk_body(ci):
                o = pl.ds(pl.multiple_of(ci * L, L), L)
                a_buf[r, o] = (a_buf[r, o] * wv0 + b_buf[r, o] * wv1
                               + s_buf[r, o] * wvg)
        pltpu.sync_copy(a_buf, out_hbm.at[pl.ds(base, rows)])
        return 0

    lax.fori_loop(0, n_rounds, round_body, 0)


def kernel(hidden_states, gate_w, expert_gate_up_w, expert_down_w,
           shared_gate_up_w, shared_down_w, shared_expert_gate_w):
    T, H = hidden_states.shape
    E = gate_w.shape[0]
    I2 = expert_gate_up_w.shape[1]
    I = I2 // 2
    K = 2
    NRT = (T * K) // BT + E
    NRS = NRT * BT
    NST = T // BT
    N_TILES = NRT + NST
    NTE = ((N_TILES + 7) // 8) * 8
    x = hidden_states.reshape(T, H)

    rw = jnp.concatenate([gate_w, shared_expert_gate_w,
                          jnp.zeros((2 * L - E - 1, H), jnp.float32)], axis=0)
    meta_i, meta_f = pl.pallas_call(
        functools.partial(_router_body, T=T, E=E, NRT=NRT, NTE=NTE),
        out_shape=(jax.ShapeDtypeStruct((T, 8), jnp.int32),
                   jax.ShapeDtypeStruct((T, 3 * L), jnp.float32)),
    )(x, rw)
    pos0 = meta_i[:, 0]
    pos1 = meta_i[:, 1]
    te = meta_i[:NTE, 2]

    x_bf = x.astype(jnp.bfloat16)
    x_i32 = lax.bitcast_convert_type(x_bf.reshape(T, H // 2, 2), jnp.int32)
    chunk = T // NW
    xs_i32 = pl.kernel(
        functools.partial(_dispatch_body, chunk=chunk),
        out_type=jax.ShapeDtypeStruct((NRS, H // 2), jnp.int32),
        mesh=plsc.VectorSubcoreMesh(core_axis_name="c", subcore_axis_name="s"),
        scratch_types=[
            pltpu.VMEM((chunk, H // 2), jnp.int32),
            pltpu.VMEM((chunk,), jnp.int32),
            pltpu.VMEM((chunk,), jnp.int32),
            pltpu.SemaphoreType.DMA,
            pltpu.SemaphoreType.DMA,
        ],
    )(x_i32, pos0, pos1)
    x_sorted = lax.bitcast_convert_type(xs_i32, jnp.bfloat16).reshape(NRS, H)

    wgu = jnp.concatenate([expert_gate_up_w, shared_gate_up_w[None]],
                          axis=0).astype(jnp.bfloat16)
    wd = jnp.concatenate([expert_down_w, shared_down_w[None]],
                         axis=0).astype(jnp.bfloat16)
    out_all = pl.pallas_call(
        functools.partial(_ffn_body, NRT=NRT, I=I),
        grid_spec=pltpu.PrefetchScalarGridSpec(
            num_scalar_prefetch=1,
            grid=(N_TILES,),
            in_specs=[
                pl.BlockSpec((BT, H), lambda i, s: (jnp.minimum(i, NRT - 1), 0)),
                pl.BlockSpec((BT, H), lambda i, s: (jnp.maximum(i - NRT, 0), 0)),
                pl.BlockSpec((1, I2, H),
                             lambda i, s: (jnp.where(s[i] < 0, E, s[i]), 0, 0)),
                pl.BlockSpec((1, H, I),
                             lambda i, s: (jnp.where(s[i] < 0, E, s[i]), 0, 0)),
            ],
            out_specs=pl.BlockSpec((BT, H), lambda i, s: (i, 0)),
        ),
        out_shape=jax.ShapeDtypeStruct((N_TILES * BT, H), jnp.float32),
    )(te, x_sorted, x_bf, wgu, wd)

    rows = 32
    n_rounds = chunk // rows
    final = pl.kernel(
        functools.partial(_combine_body, rows=rows, n_rounds=n_rounds,
                          shared_base=NRS, H=H),
        out_type=jax.ShapeDtypeStruct((T, H), jnp.float32),
        mesh=plsc.VectorSubcoreMesh(core_axis_name="c", subcore_axis_name="s"),
        scratch_types=[
            pltpu.VMEM((rows, H), jnp.float32),
            pltpu.VMEM((rows, H), jnp.float32),
            pltpu.VMEM((rows, H), jnp.float32),
            pltpu.VMEM((rows, 3 * L), jnp.float32),
            pltpu.VMEM((rows,), jnp.int32),
            pltpu.VMEM((rows,), jnp.int32),
            pltpu.SemaphoreType.DMA,
            pltpu.SemaphoreType.DMA,
        ],
    )(out_all, pos0, pos1, meta_f)

    return final.reshape(hidden_states.shape)


# speedup vs baseline: 1.7494x; 1.7494x over previous
import functools

import jax
import jax.numpy as jnp
from jax import lax
from jax.experimental import pallas as pl
from jax.experimental.pallas import tpu as pltpu
from jax.experimental.pallas import tpu_sc as plsc

NC, NS, L = 2, 16, 16
NW = NC * NS
BT = 128


def _shift_down(a, sh):
    z = jnp.zeros((sh,) + a.shape[1:], a.dtype)
    return jnp.concatenate([z, a[:-sh]], axis=0)


def _shift_right(a, sh):
    z = jnp.zeros(a.shape[:1] + (sh,) + a.shape[2:], a.dtype)
    return jnp.concatenate([z, a[:, :-sh]], axis=1)


def _router_body(x_ref, rw_ref, meta_i_ref, meta_f_ref, *, T, E, NRT, NTE):
    x = x_ref[...]
    logits = lax.dot_general(x, rw_ref[...], (((1,), (1,)), ((), ())),
                             preferred_element_type=jnp.float32)
    C = logits.shape[1]
    cols = lax.broadcasted_iota(jnp.int32, (T, C), 1)
    is_e = cols < E
    el = jnp.where(is_e, logits, -1e30)
    m = jnp.max(el, axis=1, keepdims=True)
    p = jnp.where(is_e, jnp.exp(el - m), 0.0)
    p = p / jnp.sum(p, axis=1, keepdims=True)

    p1 = jnp.max(p, axis=1, keepdims=True)
    a1 = jnp.min(jnp.where(p == p1, cols, C), axis=1, keepdims=True)
    p_wo = jnp.where(cols == a1, -1.0, p)
    p2 = jnp.max(p_wo, axis=1, keepdims=True)
    a2 = jnp.min(jnp.where(p_wo == p2, cols, C), axis=1, keepdims=True)
    wsum = p1 + p2
    w1, w2 = p1 / wsum, p2 / wsum
    g = 1.0 / (1.0 + jnp.exp(-logits[:, E:E + 1]))

    m0 = (cols == a1)
    m1 = (cols == a2)
    mm = (m0 | m1).astype(jnp.int32)
    c = mm
    sh = 1
    while sh < T:
        c = c + _shift_down(c, sh)
        sh *= 2
    counts = c[T - 1:T, :]
    excl = c - mm
    rank0 = jnp.sum(jnp.where(m0, excl, 0), axis=1, keepdims=True)
    rank1 = jnp.sum(jnp.where(m1, excl, 0), axis=1, keepdims=True)

    nt = (counts + (BT - 1)) // BT
    ts = nt
    sh = 1
    while sh < C:
        ts = ts + _shift_right(ts, sh)
        sh *= 2
    tile_start = ts - nt
    off = tile_start * BT
    pos0 = jnp.sum(jnp.where(m0, off, 0), axis=1, keepdims=True) + rank0
    pos1 = jnp.sum(jnp.where(m1, off, 0), axis=1, keepdims=True) + rank1

    ti = lax.broadcasted_iota(jnp.int32, (NTE, C), 0)
    tcols = lax.broadcasted_iota(jnp.int32, (NTE, C), 1)
    ts_b = jnp.broadcast_to(tile_start, (NTE, C))
    nt_b = jnp.broadcast_to(nt, (NTE, C))
    ind = ((ti >= ts_b) & (ti < ts_b + nt_b) & (tcols < E)).astype(jnp.int32)
    any_ind = jnp.sum(ind, axis=1, keepdims=True)
    te = jnp.sum(ind * tcols, axis=1, keepdims=True) - (1 - any_ind)
    te = jnp.where(ti[:, :1] >= NRT, E, te)

    meta_i_ref[:, 0:1] = pos0
    meta_i_ref[:, 1:2] = pos1
    meta_i_ref[0:NTE, 2:3] = te
    meta_f_ref[:, 0:L] = jnp.broadcast_to(w1, (T, L))
    meta_f_ref[:, L:2 * L] = jnp.broadcast_to(w2, (T, L))
    meta_f_ref[:, 2 * L:3 * L] = jnp.broadcast_to(g, (T, L))


def _dispatch_body(x_hbm, pos0_hbm, pos1_hbm, xs_hbm, buf, idx0, idx1, sem0, sem1,
                   *, chunk):
    wid = lax.axis_index("s") * NC + lax.axis_index("c")
    base = pl.multiple_of(wid * chunk, 8)
    pltpu.sync_copy(x_hbm.at[pl.ds(base, chunk)], buf)
    pltpu.sync_copy(pos0_hbm.at[pl.ds(base, chunk)], idx0)
    pltpu.sync_copy(pos1_hbm.at[pl.ds(base, chunk)], idx1)
    c0 = pltpu.async_copy(buf, xs_hbm.at[idx0], sem0)
    c1 = pltpu.async_copy(buf, xs_hbm.at[idx1], sem1)
    c0.wait()
    c1.wait()


def _ffn_body(te_ref, xs_ref, x_ref, wgu_ref, wd_ref, o_ref, *, NRT, I):
    i = pl.program_id(0)
    tev = te_ref[i]
    rows = jnp.where(i < NRT, xs_ref[...], x_ref[...]).astype(wgu_ref.dtype)

    @pl.when(tev >= 0)
    def _():
        gu = lax.dot_general(rows, wgu_ref[0], (((1,), (1,)), ((), ())),
                             preferred_element_type=jnp.float32)
        gt = gu[:, :I]
        up = gu[:, I:]
        act = gt * (1.0 / (1.0 + jnp.exp(-gt))) * up
        o_ref[...] = lax.dot_general(act.astype(wd_ref.dtype), wd_ref[0],
                                     (((1,), (1,)), ((), ())),
                                     preferred_element_type=jnp.float32)


def _combine_body(oall_hbm, pos0_hbm, pos1_hbm, mf_hbm, out_hbm,
                  a_buf, b_buf, s_buf, w_buf, idx0, idx1, sem0, sem1,
                  *, rows, n_rounds, shared_base, H):
    wid = lax.axis_index("s") * NC + lax.axis_index("c")
    nch = H // L

    def round_body(rnd, _):
        base = pl.multiple_of(wid * (rows * n_rounds) + rnd * rows, 8)
        pltpu.sync_copy(pos0_hbm.at[pl.ds(base, rows)], idx0)
        pltpu.sync_copy(pos1_hbm.at[pl.ds(base, rows)], idx1)
        c0 = pltpu.async_copy(oall_hbm.at[idx0], a_buf, sem0)
        c1 = pltpu.async_copy(oall_hbm.at[idx1], b_buf, sem1)
        pltpu.sync_copy(oall_hbm.at[pl.ds(shared_base + base, rows)], s_buf)
        pltpu.sync_copy(mf_hbm.at[pl.ds(base, rows)], w_buf)
        c0.wait()
        c1.wait()

        @plsc.parallel_loop(0, rows)
        def row_body(r):
            wv0 = w_buf[r, pl.ds(0, L)]
            wv1 = w_buf[r, pl.ds(L, L)]
            wvg = w_buf[r, pl.ds(2 * L, L)]

            @plsc.parallel_loop(0, nch, unroll=8)
            def chunk_body(ci):
                o = pl.ds(pl.multiple_of(ci * L, L), L)
                a_buf[r, o] = (a_buf[r, o] * wv0 + b_buf[r, o] * wv1
                               + s_buf[r, o] * wvg)
        pltpu.sync_copy(a_buf, out_hbm.at[pl.ds(base, rows)])
        return 0

    lax.fori_loop(0, n_rounds, round_body, 0)


def kernel(hidden_states, gate_w, expert_gate_up_w, expert_down_w,
           shared_gate_up_w, shared_down_w, shared_expert_gate_w):
    T, H = hidden_states.shape
    E = gate_w.shape[0]
    I2 = expert_gate_up_w.shape[1]
    I = I2 // 2
    K = 2
    NRT = (T * K) // BT + E
    NRS = NRT * BT
    NST = T // BT
    N_TILES = NRT + NST
    NTE = ((N_TILES + 7) // 8) * 8
    x = hidden_states.reshape(T, H)

    rw = jnp.concatenate([gate_w, shared_expert_gate_w,
                          jnp.zeros((2 * L - E - 1, H), jnp.float32)], axis=0)
    meta_i, meta_f = pl.pallas_call(
        functools.partial(_router_body, T=T, E=E, NRT=NRT, NTE=NTE),
        out_shape=(jax.ShapeDtypeStruct((T, 8), jnp.int32),
                   jax.ShapeDtypeStruct((T, 3 * L), jnp.float32)),
    )(x, rw)
    pos0 = meta_i[:, 0]
    pos1 = meta_i[:, 1]
    te = meta_i[:NTE, 2]

    chunk = T // NW
    x_sorted = pl.kernel(
        functools.partial(_dispatch_body, chunk=chunk),
        out_type=jax.ShapeDtypeStruct((NRS, H), jnp.float32),
        mesh=plsc.VectorSubcoreMesh(core_axis_name="c", subcore_axis_name="s"),
        scratch_types=[
            pltpu.VMEM((chunk, H), jnp.float32),
            pltpu.VMEM((chunk,), jnp.int32),
            pltpu.VMEM((chunk,), jnp.int32),
            pltpu.SemaphoreType.DMA,
            pltpu.SemaphoreType.DMA,
        ],
    )(x, pos0, pos1)

    wgu = jnp.concatenate([expert_gate_up_w, shared_gate_up_w[None]],
                          axis=0).astype(jnp.bfloat16)
    wd = jnp.concatenate([expert_down_w, shared_down_w[None]],
                         axis=0).astype(jnp.bfloat16)
    out_all = pl.pallas_call(
        functools.partial(_ffn_body, NRT=NRT, I=I),
        grid_spec=pltpu.PrefetchScalarGridSpec(
            num_scalar_prefetch=1,
            grid=(N_TILES,),
            in_specs=[
                pl.BlockSpec((BT, H), lambda i, s: (jnp.minimum(i, NRT - 1), 0)),
                pl.BlockSpec((BT, H), lambda i, s: (jnp.maximum(i - NRT, 0), 0)),
                pl.BlockSpec((1, I2, H),
                             lambda i, s: (jnp.where(s[i] < 0, E, s[i]), 0, 0)),
                pl.BlockSpec((1, H, I),
                             lambda i, s: (jnp.where(s[i] < 0, E, s[i]), 0, 0)),
            ],
            out_specs=pl.BlockSpec((BT, H), lambda i, s: (i, 0)),
        ),
        out_shape=jax.ShapeDtypeStruct((N_TILES * BT, H), jnp.float32),
    )(te, x_sorted, x, wgu, wd)

    rows = 32
    n_rounds = chunk // rows
    final = pl.kernel(
        functools.partial(_combine_body, rows=rows, n_rounds=n_rounds,
                          shared_base=NRS, H=H),
        out_type=jax.ShapeDtypeStruct((T, H), jnp.float32),
        mesh=plsc.VectorSubcoreMesh(core_axis_name="c", subcore_axis_name="s"),
        scratch_types=[
            pltpu.VMEM((rows, H), jnp.float32),
            pltpu.VMEM((rows, H), jnp.float32),
            pltpu.VMEM((rows, H), jnp.float32),
            pltpu.VMEM((rows, 3 * L), jnp.float32),
            pltpu.VMEM((rows,), jnp.int32),
            pltpu.VMEM((rows,), jnp.int32),
            pltpu.SemaphoreType.DMA,
            pltpu.SemaphoreType.DMA,
        ],
    )(out_all, pos0, pos1, meta_f)

    return final.reshape(hidden_states.shape)


# speedup vs baseline: 2.4298x; 1.3890x over previous
import functools

import jax
import jax.numpy as jnp
from jax import lax
from jax.experimental import pallas as pl
from jax.experimental.pallas import tpu as pltpu
from jax.experimental.pallas import tpu_sc as plsc

NC, NS, L = 2, 16, 16
NW = NC * NS
BT = 128
BS = 256


def _shift_down(a, sh):
    z = jnp.zeros((sh,) + a.shape[1:], a.dtype)
    return jnp.concatenate([z, a[:-sh]], axis=0)


def _shift_right(a, sh):
    z = jnp.zeros(a.shape[:1] + (sh,) + a.shape[2:], a.dtype)
    return jnp.concatenate([z, a[:, :-sh]], axis=1)


def _router_body(x_ref, rw_ref, meta_i_ref, meta_f_ref, *, T, E, NRT, NTE):
    x = x_ref[...]
    logits = lax.dot_general(x, rw_ref[...], (((1,), (1,)), ((), ())),
                             preferred_element_type=jnp.float32)
    C = logits.shape[1]
    cols = lax.broadcasted_iota(jnp.int32, (T, C), 1)
    is_e = cols < E
    el = jnp.where(is_e, logits, -1e30)
    m = jnp.max(el, axis=1, keepdims=True)
    p = jnp.where(is_e, jnp.exp(el - m), 0.0)
    p = p / jnp.sum(p, axis=1, keepdims=True)

    p1 = jnp.max(p, axis=1, keepdims=True)
    a1 = jnp.min(jnp.where(p == p1, cols, C), axis=1, keepdims=True)
    p_wo = jnp.where(cols == a1, -1.0, p)
    p2 = jnp.max(p_wo, axis=1, keepdims=True)
    a2 = jnp.min(jnp.where(p_wo == p2, cols, C), axis=1, keepdims=True)
    wsum = p1 + p2
    w1, w2 = p1 / wsum, p2 / wsum
    g = 1.0 / (1.0 + jnp.exp(-logits[:, E:E + 1]))

    m0 = (cols == a1)
    m1 = (cols == a2)
    mm = (m0 | m1).astype(jnp.int32)
    c = mm
    sh = 1
    while sh < T:
        c = c + _shift_down(c, sh)
        sh *= 2
    counts = c[T - 1:T, :]
    excl = c - mm
    rank0 = jnp.sum(jnp.where(m0, excl, 0), axis=1, keepdims=True)
    rank1 = jnp.sum(jnp.where(m1, excl, 0), axis=1, keepdims=True)

    nt = (counts + (BT - 1)) // BT
    ts = nt
    sh = 1
    while sh < C:
        ts = ts + _shift_right(ts, sh)
        sh *= 2
    tile_start = ts - nt
    off = tile_start * BT
    pos0 = jnp.sum(jnp.where(m0, off, 0), axis=1, keepdims=True) + rank0
    pos1 = jnp.sum(jnp.where(m1, off, 0), axis=1, keepdims=True) + rank1

    ti = lax.broadcasted_iota(jnp.int32, (NTE, C), 0)
    tcols = lax.broadcasted_iota(jnp.int32, (NTE, C), 1)
    ts_b = jnp.broadcast_to(tile_start, (NTE, C))
    nt_b = jnp.broadcast_to(nt, (NTE, C))
    ind = ((ti >= ts_b) & (ti < ts_b + nt_b) & (tcols < E)).astype(jnp.int32)
    any_ind = jnp.sum(ind, axis=1, keepdims=True)
    te = jnp.sum(ind * tcols, axis=1, keepdims=True) - (1 - any_ind)

    meta_i_ref[:, 0:1] = pos0
    meta_i_ref[:, 1:2] = pos1
    meta_i_ref[0:NTE, 2:3] = te
    meta_f_ref[:, 0:L] = jnp.broadcast_to(w1, (T, L))
    meta_f_ref[:, L:2 * L] = jnp.broadcast_to(w2, (T, L))
    meta_f_ref[:, 2 * L:3 * L] = jnp.broadcast_to(g, (T, L))


def _dispatch_body(x_hbm, pos0_hbm, pos1_hbm, xs_hbm, buf, idx0, idx1, sem0, sem1,
                   *, chunk):
    wid = lax.axis_index("s") * NC + lax.axis_index("c")
    base = pl.multiple_of(wid * chunk, 8)
    pltpu.sync_copy(x_hbm.at[pl.ds(base, chunk)], buf)
    pltpu.sync_copy(pos0_hbm.at[pl.ds(base, chunk)], idx0)
    pltpu.sync_copy(pos1_hbm.at[pl.ds(base, chunk)], idx1)
    c0 = pltpu.async_copy(buf, xs_hbm.at[idx0], sem0)
    c1 = pltpu.async_copy(buf, xs_hbm.at[idx1], sem1)
    c0.wait()
    c1.wait()


def _shared_body(x_ref, wgu_ref, wd_ref, o_ref, *, I):
    gu = lax.dot_general(x_ref[...], wgu_ref[...], (((1,), (1,)), ((), ())),
                         preferred_element_type=jnp.float32)
    gt = gu[:, :I]
    up = gu[:, I:]
    act = gt * (1.0 / (1.0 + jnp.exp(-gt))) * up
    o_ref[...] = lax.dot_general(act, wd_ref[...], (((1,), (1,)), ((), ())),
                                 preferred_element_type=jnp.float32)


def _ffn_body(te_ref, xs_ref, wgu_ref, wd_ref, o_ref, *, I):
    tev = te_ref[pl.program_id(0)]

    @pl.when(tev >= 0)
    def _():
        gu = lax.dot_general(xs_ref[...], wgu_ref[0], (((1,), (1,)), ((), ())),
                             preferred_element_type=jnp.float32)
        gt = gu[:, :I]
        up = gu[:, I:]
        act = gt * (1.0 / (1.0 + jnp.exp(-gt))) * up
        o_ref[...] = lax.dot_general(act, wd_ref[0], (((1,), (1,)), ((), ())),
                                     preferred_element_type=jnp.float32)


def _combine_body(orouted_hbm, oshared_hbm, pos0_hbm, pos1_hbm, mf_hbm, out_hbm,
                  a_buf, b_buf, s_buf, w_buf, idx0, idx1, sem0, sem1,
                  *, rows, n_rounds, H):
    wid = lax.axis_index("s") * NC + lax.axis_index("c")
    nch = H // L

    def round_body(rnd, _):
        base = pl.multiple_of(wid * (rows * n_rounds) + rnd * rows, 8)
        pltpu.sync_copy(pos0_hbm.at[pl.ds(base, rows)], idx0)
        pltpu.sync_copy(pos1_hbm.at[pl.ds(base, rows)], idx1)
        c0 = pltpu.async_copy(orouted_hbm.at[idx0], a_buf, sem0)
        c1 = pltpu.async_copy(orouted_hbm.at[idx1], b_buf, sem1)
        pltpu.sync_copy(oshared_hbm.at[pl.ds(base, rows)], s_buf)
        pltpu.sync_copy(mf_hbm.at[pl.ds(base, rows)], w_buf)
        c0.wait()
        c1.wait()

        @plsc.parallel_loop(0, rows)
        def row_body(r):
            wv0 = w_buf[r, pl.ds(0, L)]
            wv1 = w_buf[r, pl.ds(L, L)]
            wvg = w_buf[r, pl.ds(2 * L, L)]

            @plsc.parallel_loop(0, nch, unroll=8)
            def chunk_body(ci):
                o = pl.ds(pl.multiple_of(ci * L, L), L)
                a_buf[r, o] = (a_buf[r, o] * wv0 + b_buf[r, o] * wv1
                               + s_buf[r, o] * wvg)

        pltpu.sync_copy(a_buf, out_hbm.at[pl.ds(base, rows)])
        return 0

    lax.fori_loop(0, n_rounds, round_body, 0)


def kernel(hidden_states, gate_w, expert_gate_up_w, expert_down_w,
           shared_gate_up_w, shared_down_w, shared_expert_gate_w):
    T, H = hidden_states.shape
    E = gate_w.shape[0]
    I2 = expert_gate_up_w.shape[1]
    I = I2 // 2
    K = 2
    NRT = (T * K) // BT + E
    NRS = NRT * BT
    NTE = ((NRT + 7) // 8) * 8
    x = hidden_states.reshape(T, H)

    rw = jnp.concatenate([gate_w, shared_expert_gate_w,
                          jnp.zeros((2 * L - E - 1, H), jnp.float32)], axis=0)
    meta_i, meta_f = pl.pallas_call(
        functools.partial(_router_body, T=T, E=E, NRT=NRT, NTE=NTE),
        out_shape=(jax.ShapeDtypeStruct((T, 8), jnp.int32),
                   jax.ShapeDtypeStruct((T, 3 * L), jnp.float32)),
    )(x, rw)
    pos0 = meta_i[:, 0]
    pos1 = meta_i[:, 1]
    te = meta_i[:NTE, 2]

    chunk = T // NW
    x_sorted = pl.kernel(
        functools.partial(_dispatch_body, chunk=chunk),
        out_type=jax.ShapeDtypeStruct((NRS, H), jnp.float32),
        mesh=plsc.VectorSubcoreMesh(core_axis_name="c", subcore_axis_name="s"),
        scratch_types=[
            pltpu.VMEM((chunk, H), jnp.float32),
            pltpu.VMEM((chunk,), jnp.int32),
            pltpu.VMEM((chunk,), jnp.int32),
            pltpu.SemaphoreType.DMA,
            pltpu.SemaphoreType.DMA,
        ],
    )(x, pos0, pos1)

    out_shared = pl.pallas_call(
        functools.partial(_shared_body, I=I),
        grid=(T // BS,),
        in_specs=[
            pl.BlockSpec((BS, H), lambda i: (i, 0)),
            pl.BlockSpec((I2, H), lambda i: (0, 0)),
            pl.BlockSpec((H, I), lambda i: (0, 0)),
        ],
        out_specs=pl.BlockSpec((BS, H), lambda i: (i, 0)),
        out_shape=jax.ShapeDtypeStruct((T, H), jnp.float32),
    )(x, shared_gate_up_w, shared_down_w)

    out_routed = pl.pallas_call(
        functools.partial(_ffn_body, I=I),
        grid_spec=pltpu.PrefetchScalarGridSpec(
            num_scalar_prefetch=1,
            grid=(NRT,),
            in_specs=[
                pl.BlockSpec((BT, H), lambda i, s: (i, 0)),
                pl.BlockSpec((1, I2, H),
                             lambda i, s, E=E: (jnp.where(s[i] < 0, E - 1, s[i]), 0, 0)),
                pl.BlockSpec((1, H, I),
                             lambda i, s, E=E: (jnp.where(s[i] < 0, E - 1, s[i]), 0, 0)),
            ],
            out_specs=pl.BlockSpec((BT, H), lambda i, s: (i, 0)),
        ),
        out_shape=jax.ShapeDtypeStruct((NRS, H), jnp.float32),
    )(te, x_sorted, expert_gate_up_w, expert_down_w)

    rows = 32
    n_rounds = chunk // rows
    final = pl.kernel(
        functools.partial(_combine_body, rows=rows, n_rounds=n_rounds, H=H),
        out_type=jax.ShapeDtypeStruct((T, H), jnp.float32),
        mesh=plsc.VectorSubcoreMesh(core_axis_name="c", subcore_axis_name="s"),
        scratch_types=[
            pltpu.VMEM((rows, H), jnp.float32),
            pltpu.VMEM((rows, H), jnp.float32),
            pltpu.VMEM((rows, H), jnp.float32),
            pltpu.VMEM((rows, 3 * L), jnp.float32),
            pltpu.VMEM((rows,), jnp.int32),
            pltpu.VMEM((rows,), jnp.int32),
            pltpu.SemaphoreType.DMA,
            pltpu.SemaphoreType.DMA,
        ],
    )(out_routed, out_shared, pos0, pos1, meta_f)

    return final.reshape(hidden_states.shape)


# speedup vs baseline: 2.7858x; 1.1465x over previous
import functools

import jax
import jax.numpy as jnp
from jax import lax
from jax.experimental import pallas as pl
from jax.experimental.pallas import tpu as pltpu
from jax.experimental.pallas import tpu_sc as plsc

NC, NS, L = 2, 16, 16
NW = NC * NS
BT = 256
BS = 256


def _shift_down(a, sh):
    z = jnp.zeros((sh,) + a.shape[1:], a.dtype)
    return jnp.concatenate([z, a[:-sh]], axis=0)


def _shift_right(a, sh):
    z = jnp.zeros(a.shape[:1] + (sh,) + a.shape[2:], a.dtype)
    return jnp.concatenate([z, a[:, :-sh]], axis=1)


def _router_body(x_ref, rw_ref, meta_i_ref, meta_f_ref, *, T, E, NRT, NTE):
    x = x_ref[...]
    logits = lax.dot_general(x, rw_ref[...], (((1,), (1,)), ((), ())),
                             preferred_element_type=jnp.float32)
    C = logits.shape[1]
    cols = lax.broadcasted_iota(jnp.int32, (T, C), 1)
    is_e = cols < E
    el = jnp.where(is_e, logits, -1e30)
    m = jnp.max(el, axis=1, keepdims=True)
    p = jnp.where(is_e, jnp.exp(el - m), 0.0)
    p = p / jnp.sum(p, axis=1, keepdims=True)

    p1 = jnp.max(p, axis=1, keepdims=True)
    a1 = jnp.min(jnp.where(p == p1, cols, C), axis=1, keepdims=True)
    p_wo = jnp.where(cols == a1, -1.0, p)
    p2 = jnp.max(p_wo, axis=1, keepdims=True)
    a2 = jnp.min(jnp.where(p_wo == p2, cols, C), axis=1, keepdims=True)
    wsum = p1 + p2
    w1, w2 = p1 / wsum, p2 / wsum
    g = 1.0 / (1.0 + jnp.exp(-logits[:, E:E + 1]))

    m0 = (cols == a1)
    m1 = (cols == a2)
    mm = (m0 | m1).astype(jnp.int32)
    c = mm
    sh = 1
    while sh < T:
        c = c + _shift_down(c, sh)
        sh *= 2
    counts = c[T - 1:T, :]
    excl = c - mm
    rank0 = jnp.sum(jnp.where(m0, excl, 0), axis=1, keepdims=True)
    rank1 = jnp.sum(jnp.where(m1, excl, 0), axis=1, keepdims=True)

    nt = (counts + (BT - 1)) // BT
    ts = nt
    sh = 1
    while sh < C:
        ts = ts + _shift_right(ts, sh)
        sh *= 2
    tile_start = ts - nt
    off = tile_start * BT
    pos0 = jnp.sum(jnp.where(m0, off, 0), axis=1, keepdims=True) + rank0
    pos1 = jnp.sum(jnp.where(m1, off, 0), axis=1, keepdims=True) + rank1

    ti = lax.broadcasted_iota(jnp.int32, (NTE, C), 0)
    tcols = lax.broadcasted_iota(jnp.int32, (NTE, C), 1)
    ts_b = jnp.broadcast_to(tile_start, (NTE, C))
    nt_b = jnp.broadcast_to(nt, (NTE, C))
    ind = ((ti >= ts_b) & (ti < ts_b + nt_b) & (tcols < E)).astype(jnp.int32)
    any_ind = jnp.sum(ind, axis=1, keepdims=True)
    te = jnp.sum(ind * tcols, axis=1, keepdims=True) - (1 - any_ind)

    meta_i_ref[:, 0:1] = pos0
    meta_i_ref[:, 1:2] = pos1
    meta_i_ref[0:NTE, 2:3] = te
    meta_f_ref[:, 0:L] = jnp.broadcast_to(w1, (T, L))
    meta_f_ref[:, L:2 * L] = jnp.broadcast_to(w2, (T, L))
    meta_f_ref[:, 2 * L:3 * L] = jnp.broadcast_to(g, (T, L))


def _dispatch_body(x_hbm, pos0_hbm, pos1_hbm, xs_hbm, buf, idx0, idx1, sem0, sem1,
                   *, chunk):
    wid = lax.axis_index("s") * NC + lax.axis_index("c")
    base = pl.multiple_of(wid * chunk, 8)
    pltpu.sync_copy(x_hbm.at[pl.ds(base, chunk)], buf)
    pltpu.sync_copy(pos0_hbm.at[pl.ds(base, chunk)], idx0)
    pltpu.sync_copy(pos1_hbm.at[pl.ds(base, chunk)], idx1)
    c0 = pltpu.async_copy(buf, xs_hbm.at[idx0], sem0)
    c1 = pltpu.async_copy(buf, xs_hbm.at[idx1], sem1)
    c0.wait()
    c1.wait()


def _shared_body(x_ref, wgu_ref, wd_ref, o_ref, *, I):
    gu = lax.dot_general(x_ref[...], wgu_ref[...], (((1,), (1,)), ((), ())),
                         preferred_element_type=jnp.float32)
    gt = gu[:, :I]
    up = gu[:, I:]
    act = gt * (1.0 / (1.0 + jnp.exp(-gt))) * up
    o_ref[...] = lax.dot_general(act, wd_ref[...], (((1,), (1,)), ((), ())),
                                 preferred_element_type=jnp.float32)


def _ffn_body(te_ref, xs_ref, wgu_ref, wd_ref, o_ref, *, I):
    tev = te_ref[pl.program_id(0)]

    @pl.when(tev >= 0)
    def _():
        gu = lax.dot_general(xs_ref[...], wgu_ref[0], (((1,), (1,)), ((), ())),
                             preferred_element_type=jnp.float32)
        gt = gu[:, :I]
        up = gu[:, I:]
        act = gt * (1.0 / (1.0 + jnp.exp(-gt))) * up
        o_ref[...] = lax.dot_general(act, wd_ref[0], (((1,), (1,)), ((), ())),
                                     preferred_element_type=jnp.float32)


def _combine_body(orouted_hbm, oshared_hbm, pos0_hbm, pos1_hbm, mf_hbm, out_hbm,
                  a_buf, b_buf, s_buf, w_buf, idx0, idx1, sem0, sem1,
                  *, rows, n_rounds, H):
    wid = lax.axis_index("s") * NC + lax.axis_index("c")
    nch = H // L

    def round_body(rnd, _):
        base = pl.multiple_of(wid * (rows * n_rounds) + rnd * rows, 8)
        pltpu.sync_copy(pos0_hbm.at[pl.ds(base, rows)], idx0)
        pltpu.sync_copy(pos1_hbm.at[pl.ds(base, rows)], idx1)
        c0 = pltpu.async_copy(orouted_hbm.at[idx0], a_buf, sem0)
        c1 = pltpu.async_copy(orouted_hbm.at[idx1], b_buf, sem1)
        pltpu.sync_copy(oshared_hbm.at[pl.ds(base, rows)], s_buf)
        pltpu.sync_copy(mf_hbm.at[pl.ds(base, rows)], w_buf)
        c0.wait()
        c1.wait()

        @plsc.parallel_loop(0, rows)
        def row_body(r):
            wv0 = w_buf[r, pl.ds(0, L)]
            wv1 = w_buf[r, pl.ds(L, L)]
            wvg = w_buf[r, pl.ds(2 * L, L)]

            @plsc.parallel_loop(0, nch, unroll=8)
            def chunk_body(ci):
                o = pl.ds(pl.multiple_of(ci * L, L), L)
                a_buf[r, o] = (a_buf[r, o] * wv0 + b_buf[r, o] * wv1
                               + s_buf[r, o] * wvg)

        pltpu.sync_copy(a_buf, out_hbm.at[pl.ds(base, rows)])
        return 0

    lax.fori_loop(0, n_rounds, round_body, 0)


def kernel(hidden_states, gate_w, expert_gate_up_w, expert_down_w,
           shared_gate_up_w, shared_down_w, shared_expert_gate_w):
    T, H = hidden_states.shape
    E = gate_w.shape[0]
    I2 = expert_gate_up_w.shape[1]
    I = I2 // 2
    K = 2
    NRT = (T * K) // BT + E
    NRS = NRT * BT
    NTE = ((NRT + 7) // 8) * 8
    x = hidden_states.reshape(T, H)

    rw = jnp.concatenate([gate_w, shared_expert_gate_w,
                          jnp.zeros((2 * L - E - 1, H), jnp.float32)], axis=0)
    meta_i, meta_f = pl.pallas_call(
        functools.partial(_router_body, T=T, E=E, NRT=NRT, NTE=NTE),
        out_shape=(jax.ShapeDtypeStruct((T, 8), jnp.int32),
                   jax.ShapeDtypeStruct((T, 3 * L), jnp.float32)),
    )(x, rw)
    pos0 = meta_i[:, 0]
    pos1 = meta_i[:, 1]
    te = meta_i[:NTE, 2]

    chunk = T // NW
    x_sorted = pl.kernel(
        functools.partial(_dispatch_body, chunk=chunk),
        out_type=jax.ShapeDtypeStruct((NRS, H), jnp.float32),
        mesh=plsc.VectorSubcoreMesh(core_axis_name="c", subcore_axis_name="s"),
        scratch_types=[
            pltpu.VMEM((chunk, H), jnp.float32),
            pltpu.VMEM((chunk,), jnp.int32),
            pltpu.VMEM((chunk,), jnp.int32),
            pltpu.SemaphoreType.DMA,
            pltpu.SemaphoreType.DMA,
        ],
    )(x, pos0, pos1)

    out_shared = pl.pallas_call(
        functools.partial(_shared_body, I=I),
        grid=(T // BS,),
        in_specs=[
            pl.BlockSpec((BS, H), lambda i: (i, 0)),
            pl.BlockSpec((I2, H), lambda i: (0, 0)),
            pl.BlockSpec((H, I), lambda i: (0, 0)),
        ],
        out_specs=pl.BlockSpec((BS, H), lambda i: (i, 0)),
        out_shape=jax.ShapeDtypeStruct((T, H), jnp.float32),
    )(x, shared_gate_up_w, shared_down_w)

    out_routed = pl.pallas_call(
        functools.partial(_ffn_body, I=I),
        grid_spec=pltpu.PrefetchScalarGridSpec(
            num_scalar_prefetch=1,
            grid=(NRT,),
            in_specs=[
                pl.BlockSpec((BT, H), lambda i, s: (i, 0)),
                pl.BlockSpec((1, I2, H),
                             lambda i, s, E=E: (jnp.where(s[i] < 0, E - 1, s[i]), 0, 0)),
                pl.BlockSpec((1, H, I),
                             lambda i, s, E=E: (jnp.where(s[i] < 0, E - 1, s[i]), 0, 0)),
            ],
            out_specs=pl.BlockSpec((BT, H), lambda i, s: (i, 0)),
        ),
        out_shape=jax.ShapeDtypeStruct((NRS, H), jnp.float32),
    )(te, x_sorted, expert_gate_up_w, expert_down_w)

    rows = 32
    n_rounds = chunk // rows
    final = pl.kernel(
        functools.partial(_combine_body, rows=rows, n_rounds=n_rounds, H=H),
        out_type=jax.ShapeDtypeStruct((T, H), jnp.float32),
        mesh=plsc.VectorSubcoreMesh(core_axis_name="c", subcore_axis_name="s"),
        scratch_types=[
            pltpu.VMEM((rows, H), jnp.float32),
            pltpu.VMEM((rows, H), jnp.float32),
            pltpu.VMEM((rows, H), jnp.float32),
            pltpu.VMEM((rows, 3 * L), jnp.float32),
            pltpu.VMEM((rows,), jnp.int32),
            pltpu.VMEM((rows,), jnp.int32),
            pltpu.SemaphoreType.DMA,
            pltpu.SemaphoreType.DMA,
        ],
    )(out_routed, out_shared, pos0, pos1, meta_f)

    return final.reshape(hidden_states.shape)


# speedup vs baseline: 2.9559x; 1.0611x over previous
import functools

import jax
import jax.numpy as jnp
from jax import lax
from jax.experimental import pallas as pl
from jax.experimental.pallas import tpu as pltpu
from jax.experimental.pallas import tpu_sc as plsc

NC, NS, L = 2, 16, 16
NW = NC * NS
BT = 512
BS = 256


def _shift_down(a, sh):
    z = jnp.zeros((sh,) + a.shape[1:], a.dtype)
    return jnp.concatenate([z, a[:-sh]], axis=0)


def _shift_right(a, sh):
    z = jnp.zeros(a.shape[:1] + (sh,) + a.shape[2:], a.dtype)
    return jnp.concatenate([z, a[:, :-sh]], axis=1)


def _router_body(x_ref, rw_ref, meta_i_ref, meta_f_ref, *, T, E, NRT, NTE):
    x = x_ref[...]
    logits = lax.dot_general(x, rw_ref[...], (((1,), (1,)), ((), ())),
                             preferred_element_type=jnp.float32)
    C = logits.shape[1]
    cols = lax.broadcasted_iota(jnp.int32, (T, C), 1)
    is_e = cols < E
    el = jnp.where(is_e, logits, -1e30)
    m = jnp.max(el, axis=1, keepdims=True)
    p = jnp.where(is_e, jnp.exp(el - m), 0.0)
    p = p / jnp.sum(p, axis=1, keepdims=True)

    p1 = jnp.max(p, axis=1, keepdims=True)
    a1 = jnp.min(jnp.where(p == p1, cols, C), axis=1, keepdims=True)
    p_wo = jnp.where(cols == a1, -1.0, p)
    p2 = jnp.max(p_wo, axis=1, keepdims=True)
    a2 = jnp.min(jnp.where(p_wo == p2, cols, C), axis=1, keepdims=True)
    wsum = p1 + p2
    w1, w2 = p1 / wsum, p2 / wsum
    g = 1.0 / (1.0 + jnp.exp(-logits[:, E:E + 1]))

    m0 = (cols == a1)
    m1 = (cols == a2)
    mm = (m0 | m1).astype(jnp.int32)
    c = mm
    sh = 1
    while sh < T:
        c = c + _shift_down(c, sh)
        sh *= 2
    counts = c[T - 1:T, :]
    excl = c - mm
    rank0 = jnp.sum(jnp.where(m0, excl, 0), axis=1, keepdims=True)
    rank1 = jnp.sum(jnp.where(m1, excl, 0), axis=1, keepdims=True)

    nt = (counts + (BT - 1)) // BT
    ts = nt
    sh = 1
    while sh < C:
        ts = ts + _shift_right(ts, sh)
        sh *= 2
    tile_start = ts - nt
    off = tile_start * BT
    pos0 = jnp.sum(jnp.where(m0, off, 0), axis=1, keepdims=True) + rank0
    pos1 = jnp.sum(jnp.where(m1, off, 0), axis=1, keepdims=True) + rank1

    ti = lax.broadcasted_iota(jnp.int32, (NTE, C), 0)
    tcols = lax.broadcasted_iota(jnp.int32, (NTE, C), 1)
    ts_b = jnp.broadcast_to(tile_start, (NTE, C))
    nt_b = jnp.broadcast_to(nt, (NTE, C))
    ind = ((ti >= ts_b) & (ti < ts_b + nt_b) & (tcols < E)).astype(jnp.int32)
    any_ind = jnp.sum(ind, axis=1, keepdims=True)
    te = jnp.sum(ind * tcols, axis=1, keepdims=True) - (1 - any_ind)

    meta_i_ref[:, 0:1] = pos0
    meta_i_ref[:, 1:2] = pos1
    meta_i_ref[0:NTE, 2:3] = te
    meta_f_ref[:, 0:L] = jnp.broadcast_to(w1, (T, L))
    meta_f_ref[:, L:2 * L] = jnp.broadcast_to(w2, (T, L))
    meta_f_ref[:, 2 * L:3 * L] = jnp.broadcast_to(g, (T, L))


def _dispatch_body(x_hbm, pos0_hbm, pos1_hbm, xs_hbm, buf, idx0, idx1, sem0, sem1,
                   *, chunk):
    wid = lax.axis_index("s") * NC + lax.axis_index("c")
    base = pl.multiple_of(wid * chunk, 8)
    pltpu.sync_copy(x_hbm.at[pl.ds(base, chunk)], buf)
    pltpu.sync_copy(pos0_hbm.at[pl.ds(base, chunk)], idx0)
    pltpu.sync_copy(pos1_hbm.at[pl.ds(base, chunk)], idx1)
    c0 = pltpu.async_copy(buf, xs_hbm.at[idx0], sem0)
    c1 = pltpu.async_copy(buf, xs_hbm.at[idx1], sem1)
    c0.wait()
    c1.wait()


def _shared_body(x_ref, wgu_ref, wd_ref, o_ref, *, I):
    gu = lax.dot_general(x_ref[...], wgu_ref[...], (((1,), (1,)), ((), ())),
                         preferred_element_type=jnp.float32)
    gt = gu[:, :I]
    up = gu[:, I:]
    act = gt * (1.0 / (1.0 + jnp.exp(-gt))) * up
    o_ref[...] = lax.dot_general(act, wd_ref[...], (((1,), (1,)), ((), ())),
                                 preferred_element_type=jnp.float32)


def _ffn_body(te_ref, xs_ref, wgu_ref, wd_ref, o_ref, *, I):
    tev = te_ref[pl.program_id(0)]

    @pl.when(tev >= 0)
    def _():
        gu = lax.dot_general(xs_ref[...], wgu_ref[0], (((1,), (1,)), ((), ())),
                             preferred_element_type=jnp.float32)
        gt = gu[:, :I]
        up = gu[:, I:]
        act = gt * (1.0 / (1.0 + jnp.exp(-gt))) * up
        o_ref[...] = lax.dot_general(act, wd_ref[0], (((1,), (1,)), ((), ())),
                                     preferred_element_type=jnp.float32)


def _combine_body(orouted_hbm, oshared_hbm, pos0_hbm, pos1_hbm, mf_hbm, out_hbm,
                  a_buf, b_buf, s_buf, w_buf, idx0, idx1, sem0, sem1,
                  *, rows, n_rounds, H):
    wid = lax.axis_index("s") * NC + lax.axis_index("c")
    nch = H // L

    def round_body(rnd, _):
        base = pl.multiple_of(wid * (rows * n_rounds) + rnd * rows, 8)
        pltpu.sync_copy(pos0_hbm.at[pl.ds(base, rows)], idx0)
        pltpu.sync_copy(pos1_hbm.at[pl.ds(base, rows)], idx1)
        c0 = pltpu.async_copy(orouted_hbm.at[idx0], a_buf, sem0)
        c1 = pltpu.async_copy(orouted_hbm.at[idx1], b_buf, sem1)
        pltpu.sync_copy(oshared_hbm.at[pl.ds(base, rows)], s_buf)
        pltpu.sync_copy(mf_hbm.at[pl.ds(base, rows)], w_buf)
        c0.wait()
        c1.wait()

        @plsc.parallel_loop(0, rows)
        def row_body(r):
            wv0 = w_buf[r, pl.ds(0, L)]
            wv1 = w_buf[r, pl.ds(L, L)]
            wvg = w_buf[r, pl.ds(2 * L, L)]

            @plsc.parallel_loop(0, nch, unroll=8)
            def chunk_body(ci):
                o = pl.ds(pl.multiple_of(ci * L, L), L)
                a_buf[r, o] = (a_buf[r, o] * wv0 + b_buf[r, o] * wv1
                               + s_buf[r, o] * wvg)

        pltpu.sync_copy(a_buf, out_hbm.at[pl.ds(base, rows)])
        return 0

    lax.fori_loop(0, n_rounds, round_body, 0)


def kernel(hidden_states, gate_w, expert_gate_up_w, expert_down_w,
           shared_gate_up_w, shared_down_w, shared_expert_gate_w):
    T, H = hidden_states.shape
    E = gate_w.shape[0]
    I2 = expert_gate_up_w.shape[1]
    I = I2 // 2
    K = 2
    NRT = (T * K) // BT + E
    NRS = NRT * BT
    NTE = ((NRT + 7) // 8) * 8
    x = hidden_states.reshape(T, H)

    rw = jnp.concatenate([gate_w, shared_expert_gate_w,
                          jnp.zeros((2 * L - E - 1, H), jnp.float32)], axis=0)
    meta_i, meta_f = pl.pallas_call(
        functools.partial(_router_body, T=T, E=E, NRT=NRT, NTE=NTE),
        out_shape=(jax.ShapeDtypeStruct((T, 8), jnp.int32),
                   jax.ShapeDtypeStruct((T, 3 * L), jnp.float32)),
    )(x, rw)
    pos0 = meta_i[:, 0]
    pos1 = meta_i[:, 1]
    te = meta_i[:NTE, 2]

    chunk = T // NW
    x_sorted = pl.kernel(
        functools.partial(_dispatch_body, chunk=chunk),
        out_type=jax.ShapeDtypeStruct((NRS, H), jnp.float32),
        mesh=plsc.VectorSubcoreMesh(core_axis_name="c", subcore_axis_name="s"),
        scratch_types=[
            pltpu.VMEM((chunk, H), jnp.float32),
            pltpu.VMEM((chunk,), jnp.int32),
            pltpu.VMEM((chunk,), jnp.int32),
            pltpu.SemaphoreType.DMA,
            pltpu.SemaphoreType.DMA,
        ],
    )(x, pos0, pos1)

    out_shared = pl.pallas_call(
        functools.partial(_shared_body, I=I),
        grid=(T // BS,),
        in_specs=[
            pl.BlockSpec((BS, H), lambda i: (i, 0)),
            pl.BlockSpec((I2, H), lambda i: (0, 0)),
            pl.BlockSpec((H, I), lambda i: (0, 0)),
        ],
        out_specs=pl.BlockSpec((BS, H), lambda i: (i, 0)),
        out_shape=jax.ShapeDtypeStruct((T, H), jnp.float32),
    )(x, shared_gate_up_w, shared_down_w)

    out_routed = pl.pallas_call(
        functools.partial(_ffn_body, I=I),
        grid_spec=pltpu.PrefetchScalarGridSpec(
            num_scalar_prefetch=1,
            grid=(NRT,),
            in_specs=[
                pl.BlockSpec((BT, H), lambda i, s: (i, 0)),
                pl.BlockSpec((1, I2, H),
                             lambda i, s, E=E: (jnp.where(s[i] < 0, E - 1, s[i]), 0, 0)),
                pl.BlockSpec((1, H, I),
                             lambda i, s, E=E: (jnp.where(s[i] < 0, E - 1, s[i]), 0, 0)),
            ],
            out_specs=pl.BlockSpec((BT, H), lambda i, s: (i, 0)),
        ),
        out_shape=jax.ShapeDtypeStruct((NRS, H), jnp.float32),
    )(te, x_sorted, expert_gate_up_w, expert_down_w)

    rows = 32
    n_rounds = chunk // rows
    final = pl.kernel(
        functools.partial(_combine_body, rows=rows, n_rounds=n_rounds, H=H),
        out_type=jax.ShapeDtypeStruct((T, H), jnp.float32),
        mesh=plsc.VectorSubcoreMesh(core_axis_name="c", subcore_axis_name="s"),
        scratch_types=[
            pltpu.VMEM((rows, H), jnp.float32),
            pltpu.VMEM((rows, H), jnp.float32),
            pltpu.VMEM((rows, H), jnp.float32),
            pltpu.VMEM((rows, 3 * L), jnp.float32),
            pltpu.VMEM((rows,), jnp.int32),
            pltpu.VMEM((rows,), jnp.int32),
            pltpu.SemaphoreType.DMA,
            pltpu.SemaphoreType.DMA,
        ],
    )(out_routed, out_shared, pos0, pos1, meta_f)

    return final.reshape(hidden_states.shape)


# speedup vs baseline: 3.1025x; 1.0496x over previous
import functools

import jax
import jax.numpy as jnp
from jax import lax
from jax.experimental import pallas as pl
from jax.experimental.pallas import tpu as pltpu
from jax.experimental.pallas import tpu_sc as plsc

NC, NS, L = 2, 16, 16
NW = NC * NS
BT = 512
BS = 256


def _shift_down(a, sh):
    z = jnp.zeros((sh,) + a.shape[1:], a.dtype)
    return jnp.concatenate([z, a[:-sh]], axis=0)


def _shift_right(a, sh):
    z = jnp.zeros(a.shape[:1] + (sh,) + a.shape[2:], a.dtype)
    return jnp.concatenate([z, a[:, :-sh]], axis=1)


def _router_body(x_ref, rw_ref, meta_i_ref, meta_f_ref, *, T, E, NRT, NTE):
    x = x_ref[...]
    logits = lax.dot_general(x, rw_ref[...], (((1,), (1,)), ((), ())),
                             preferred_element_type=jnp.float32)
    C = logits.shape[1]
    cols = lax.broadcasted_iota(jnp.int32, (T, C), 1)
    is_e = cols < E
    el = jnp.where(is_e, logits, -1e30)
    m = jnp.max(el, axis=1, keepdims=True)
    p = jnp.where(is_e, jnp.exp(el - m), 0.0)
    p = p / jnp.sum(p, axis=1, keepdims=True)

    p1 = jnp.max(p, axis=1, keepdims=True)
    a1 = jnp.min(jnp.where(p == p1, cols, C), axis=1, keepdims=True)
    p_wo = jnp.where(cols == a1, -1.0, p)
    p2 = jnp.max(p_wo, axis=1, keepdims=True)
    a2 = jnp.min(jnp.where(p_wo == p2, cols, C), axis=1, keepdims=True)
    wsum = p1 + p2
    w1, w2 = p1 / wsum, p2 / wsum
    g = 1.0 / (1.0 + jnp.exp(-logits[:, E:E + 1]))

    m0 = (cols == a1)
    m1 = (cols == a2)
    mm = (m0 | m1).astype(jnp.int32)
    c = mm
    sh = 1
    while sh < T:
        c = c + _shift_down(c, sh)
        sh *= 2
    counts = c[T - 1:T, :]
    excl = c - mm
    rank0 = jnp.sum(jnp.where(m0, excl, 0), axis=1, keepdims=True)
    rank1 = jnp.sum(jnp.where(m1, excl, 0), axis=1, keepdims=True)

    nt = (counts + (BT - 1)) // BT
    ts = nt
    sh = 1
    while sh < C:
        ts = ts + _shift_right(ts, sh)
        sh *= 2
    tile_start = ts - nt
    off = tile_start * BT
    pos0 = jnp.sum(jnp.where(m0, off, 0), axis=1, keepdims=True) + rank0
    pos1 = jnp.sum(jnp.where(m1, off, 0), axis=1, keepdims=True) + rank1

    ti = lax.broadcasted_iota(jnp.int32, (NTE, C), 0)
    tcols = lax.broadcasted_iota(jnp.int32, (NTE, C), 1)
    ts_b = jnp.broadcast_to(tile_start, (NTE, C))
    nt_b = jnp.broadcast_to(nt, (NTE, C))
    ind = ((ti >= ts_b) & (ti < ts_b + nt_b) & (tcols < E)).astype(jnp.int32)
    any_ind = jnp.sum(ind, axis=1, keepdims=True)
    te = jnp.sum(ind * tcols, axis=1, keepdims=True) - (1 - any_ind)

    meta_i_ref[:, 0:1] = pos0
    meta_i_ref[:, 1:2] = pos1
    meta_i_ref[0:NTE, 2:3] = te
    meta_f_ref[:, 0:L] = jnp.broadcast_to(w1, (T, L))
    meta_f_ref[:, L:2 * L] = jnp.broadcast_to(w2, (T, L))
    meta_f_ref[:, 2 * L:3 * L] = jnp.broadcast_to(g, (T, L))


def _dispatch_body(x_hbm, pos0_hbm, pos1_hbm, xs_hbm, buf, idx0, idx1, sem0, sem1,
                   *, chunk):
    wid = lax.axis_index("s") * NC + lax.axis_index("c")
    base = pl.multiple_of(wid * chunk, 8)
    pltpu.sync_copy(x_hbm.at[pl.ds(base, chunk)], buf)
    pltpu.sync_copy(pos0_hbm.at[pl.ds(base, chunk)], idx0)
    pltpu.sync_copy(pos1_hbm.at[pl.ds(base, chunk)], idx1)
    c0 = pltpu.async_copy(buf, xs_hbm.at[idx0], sem0)
    c1 = pltpu.async_copy(buf, xs_hbm.at[idx1], sem1)
    c0.wait()
    c1.wait()


def _shared_body(x_ref, wgu_ref, wd_ref, o_ref, *, I):
    gu = lax.dot_general(x_ref[...], wgu_ref[...], (((1,), (1,)), ((), ())),
                         preferred_element_type=jnp.float32)
    gt = gu[:, :I]
    up = gu[:, I:]
    act = gt * (1.0 / (1.0 + jnp.exp(-gt))) * up
    o_ref[...] = lax.dot_general(act, wd_ref[...], (((1,), (1,)), ((), ())),
                                 preferred_element_type=jnp.float32)


def _ffn_body(te_ref, xs_ref, wgu_ref, wd_ref, o_ref, *, I):
    tev = te_ref[pl.program_id(0)]

    @pl.when(tev >= 0)
    def _():
        gu = lax.dot_general(xs_ref[...], wgu_ref[0], (((1,), (1,)), ((), ())),
                             preferred_element_type=jnp.float32)
        gt = gu[:, :I]
        up = gu[:, I:]
        act = gt * (1.0 / (1.0 + jnp.exp(-gt))) * up
        o_ref[...] = lax.dot_general(act, wd_ref[0], (((1,), (1,)), ((), ())),
                                     preferred_element_type=jnp.float32)


def _combine_body(orouted_hbm, oshared_hbm, pos0_hbm, pos1_hbm, mf_hbm, out_hbm,
                  a0, a1, a2, b0, b1, s0, s1, w_buf, idx0, idx1,
                  sa0, sa1, sa2, sb0, sb1, ss0, ss1, so0, so1, so2,
                  *, rows, n_rounds, H):
    wid = lax.axis_index("s") * NC + lax.axis_index("c")
    nch = H // L
    chunk = rows * n_rounds
    gbase = pl.multiple_of(wid * chunk, 8)
    a_bufs, sa = (a0, a1, a2), (sa0, sa1, sa2)
    b_bufs, sb = (b0, b1), (sb0, sb1)
    s_bufs, ss = (s0, s1), (ss0, ss1)
    so = (so0, so1, so2)

    pltpu.sync_copy(pos0_hbm.at[pl.ds(gbase, chunk)], idx0)
    pltpu.sync_copy(pos1_hbm.at[pl.ds(gbase, chunk)], idx1)
    pltpu.sync_copy(mf_hbm.at[pl.ds(gbase, chunk)], w_buf)

    def issue(r):
        i0 = idx0[pl.ds(r * rows, rows)]
        i1 = idx1[pl.ds(r * rows, rows)]
        ca = pltpu.async_copy(orouted_hbm.at[i0], a_bufs[r % 3], sa[r % 3])
        cb = pltpu.async_copy(orouted_hbm.at[i1], b_bufs[r % 2], sb[r % 2])
        cs = pltpu.async_copy(
            oshared_hbm.at[pl.ds(pl.multiple_of(gbase + r * rows, 8), rows)],
            s_bufs[r % 2], ss[r % 2])
        return ca, cb, cs

    pend = {0: issue(0), 1: issue(1)}
    wouts = {}
    for r in range(n_rounds):
        for c in pend.pop(r):
            c.wait()
        a_buf, b_buf, s_buf = a_bufs[r % 3], b_bufs[r % 2], s_bufs[r % 2]

        @plsc.parallel_loop(0, rows)
        def row_body(rr, r=r, a_buf=a_buf, b_buf=b_buf, s_buf=s_buf):
            wv0 = w_buf[r * rows + rr, pl.ds(0, L)]
            wv1 = w_buf[r * rows + rr, pl.ds(L, L)]
            wvg = w_buf[r * rows + rr, pl.ds(2 * L, L)]

            @plsc.parallel_loop(0, nch, unroll=8)
            def chunk_body(ci):
                o = pl.ds(pl.multiple_of(ci * L, L), L)
                a_buf[rr, o] = (a_buf[rr, o] * wv0 + b_buf[rr, o] * wv1
                                + s_buf[rr, o] * wvg)

        wouts[r] = pltpu.async_copy(
            a_buf, out_hbm.at[pl.ds(pl.multiple_of(gbase + r * rows, 8), rows)],
            so[r % 3])
        if r + 2 < n_rounds:
            if r - 1 >= 0:
                wouts.pop(r - 1).wait()
            pend[r + 2] = issue(r + 2)
    for r in sorted(wouts):
        wouts[r].wait()


def kernel(hidden_states, gate_w, expert_gate_up_w, expert_down_w,
           shared_gate_up_w, shared_down_w, shared_expert_gate_w):
    T, H = hidden_states.shape
    E = gate_w.shape[0]
    I2 = expert_gate_up_w.shape[1]
    I = I2 // 2
    K = 2
    NRT = (T * K) // BT + E
    NRS = NRT * BT
    NTE = ((NRT + 7) // 8) * 8
    x = hidden_states.reshape(T, H)

    rw = jnp.concatenate([gate_w, shared_expert_gate_w,
                          jnp.zeros((2 * L - E - 1, H), jnp.float32)], axis=0)
    meta_i, meta_f = pl.pallas_call(
        functools.partial(_router_body, T=T, E=E, NRT=NRT, NTE=NTE),
        out_shape=(jax.ShapeDtypeStruct((T, 8), jnp.int32),
                   jax.ShapeDtypeStruct((T, 3 * L), jnp.float32)),
    )(x, rw)
    pos0 = meta_i[:, 0]
    pos1 = meta_i[:, 1]
    te = meta_i[:NTE, 2]

    chunk = T // NW
    x_sorted = pl.kernel(
        functools.partial(_dispatch_body, chunk=chunk),
        out_type=jax.ShapeDtypeStruct((NRS, H), jnp.float32),
        mesh=plsc.VectorSubcoreMesh(core_axis_name="c", subcore_axis_name="s"),
        scratch_types=[
            pltpu.VMEM((chunk, H), jnp.float32),
            pltpu.VMEM((chunk,), jnp.int32),
            pltpu.VMEM((chunk,), jnp.int32),
            pltpu.SemaphoreType.DMA,
            pltpu.SemaphoreType.DMA,
        ],
    )(x, pos0, pos1)

    out_shared = pl.pallas_call(
        functools.partial(_shared_body, I=I),
        grid=(T // BS,),
        in_specs=[
            pl.BlockSpec((BS, H), lambda i: (i, 0)),
            pl.BlockSpec((I2, H), lambda i: (0, 0)),
            pl.BlockSpec((H, I), lambda i: (0, 0)),
        ],
        out_specs=pl.BlockSpec((BS, H), lambda i: (i, 0)),
        out_shape=jax.ShapeDtypeStruct((T, H), jnp.float32),
    )(x, shared_gate_up_w, shared_down_w)

    out_routed = pl.pallas_call(
        functools.partial(_ffn_body, I=I),
        grid_spec=pltpu.PrefetchScalarGridSpec(
            num_scalar_prefetch=1,
            grid=(NRT,),
            in_specs=[
                pl.BlockSpec((BT, H), lambda i, s: (i, 0)),
                pl.BlockSpec((1, I2, H),
                             lambda i, s, E=E: (jnp.where(s[i] < 0, E - 1, s[i]), 0, 0)),
                pl.BlockSpec((1, H, I),
                             lambda i, s, E=E: (jnp.where(s[i] < 0, E - 1, s[i]), 0, 0)),
            ],
            out_specs=pl.BlockSpec((BT, H), lambda i, s: (i, 0)),
        ),
        out_shape=jax.ShapeDtypeStruct((NRS, H), jnp.float32),
    )(te, x_sorted, expert_gate_up_w, expert_down_w)

    rows = 16
    n_rounds = chunk // rows
    final = pl.kernel(
        functools.partial(_combine_body, rows=rows, n_rounds=n_rounds, H=H),
        out_type=jax.ShapeDtypeStruct((T, H), jnp.float32),
        mesh=plsc.VectorSubcoreMesh(core_axis_name="c", subcore_axis_name="s"),
        scratch_types=(
            [pltpu.VMEM((rows, H), jnp.float32)] * 5
            + [pltpu.VMEM((rows, H), jnp.float32)] * 2
            + [pltpu.VMEM((chunk, 3 * L), jnp.float32),
               pltpu.VMEM((chunk,), jnp.int32),
               pltpu.VMEM((chunk,), jnp.int32)]
            + [pltpu.SemaphoreType.DMA] * 10
        ),
    )(out_routed, out_shared, pos0, pos1, meta_f)

    return final.reshape(hidden_states.shape)


# speedup vs baseline: 3.4568x; 1.1142x over previous
import functools

import jax
import jax.numpy as jnp
from jax import lax
from jax.experimental import pallas as pl
from jax.experimental.pallas import tpu as pltpu
from jax.experimental.pallas import tpu_sc as plsc

NC, NS, L = 2, 16, 16
NW = NC * NS
BT = 512
BS = 512


def _shift_right(a, sh):
    z = jnp.zeros(a.shape[:1] + (sh,) + a.shape[2:], a.dtype)
    return jnp.concatenate([z, a[:, :-sh]], axis=1)


def _router_body(x_ref, gw_ref, sgw_ref, mi_ref, mf_ref, *, T, E, NRT):
    x = x_ref[...]
    lt = lax.dot_general(gw_ref[...], x, (((1,), (1,)), ((), ())),
                         preferred_element_type=jnp.float32)
    ls = lax.dot_general(sgw_ref[...], x, (((1,), (1,)), ((), ())),
                         preferred_element_type=jnp.float32)
    rows_e = lax.broadcasted_iota(jnp.int32, (E, T), 0)

    m = jnp.max(lt, axis=0, keepdims=True)
    p = jnp.exp(lt - m)
    p = p / jnp.sum(p, axis=0, keepdims=True)

    p1 = jnp.max(p, axis=0, keepdims=True)
    a1 = jnp.min(jnp.where(p == p1, rows_e, E), axis=0, keepdims=True)
    p_wo = jnp.where(rows_e == a1, -1.0, p)
    p2 = jnp.max(p_wo, axis=0, keepdims=True)
    a2 = jnp.min(jnp.where(p_wo == p2, rows_e, E), axis=0, keepdims=True)
    wsum = p1 + p2
    w1, w2 = p1 / wsum, p2 / wsum
    g = 1.0 / (1.0 + jnp.exp(-ls))

    m0 = (rows_e == a1)
    m1 = (rows_e == a2)
    mm = (m0 | m1).astype(jnp.int32)
    c = mm
    sh = 1
    while sh < T:
        c = c + _shift_right(c, sh)
        sh *= 2
    counts = c[:, T - 1:T]
    excl = c - mm
    rank0 = jnp.sum(jnp.where(m0, excl, 0), axis=0, keepdims=True)
    rank1 = jnp.sum(jnp.where(m1, excl, 0), axis=0, keepdims=True)

    nt = (counts + (BT - 1)) // BT
    ts = nt
    sh = 1
    while sh < E:
        z = jnp.zeros((sh, 1), jnp.int32)
        ts = ts + jnp.concatenate([z, ts[:-sh]], axis=0)
        sh *= 2
    tile_start = ts - nt
    off = tile_start * BT
    pos0 = jnp.sum(jnp.where(m0, off, 0), axis=0, keepdims=True) + rank0
    pos1 = jnp.sum(jnp.where(m1, off, 0), axis=0, keepdims=True) + rank1

    ti = lax.broadcasted_iota(jnp.int32, (E, NRT), 1)
    ts_b = jnp.broadcast_to(tile_start, (E, NRT))
    nt_b = jnp.broadcast_to(nt, (E, NRT))
    rows8 = lax.broadcasted_iota(jnp.int32, (E, NRT), 0)
    ind = ((ti >= ts_b) & (ti < ts_b + nt_b)).astype(jnp.int32)
    any_ind = jnp.sum(ind, axis=0, keepdims=True)
    te = jnp.sum(ind * rows8, axis=0, keepdims=True) - (1 - any_ind)

    mi_ref[0:1, :] = pos0
    mi_ref[1:2, :] = pos1
    mi_ref[2:3, 0:NRT] = te
    mf_ref[0:1, :] = w1
    mf_ref[1:2, :] = w2
    mf_ref[2:3, :] = g


def _dispatch_body(x_hbm, mi_hbm, xs_hbm, buf, idx0, idx1, sem0, sem1, *, chunk):
    wid = lax.axis_index("s") * NC + lax.axis_index("c")
    base = pl.multiple_of(wid * chunk, 8)
    pltpu.sync_copy(x_hbm.at[pl.ds(base, chunk)], buf)
    pltpu.sync_copy(mi_hbm.at[0, pl.ds(base, chunk)], idx0)
    pltpu.sync_copy(mi_hbm.at[1, pl.ds(base, chunk)], idx1)
    c0 = pltpu.async_copy(buf, xs_hbm.at[idx0], sem0)
    c1 = pltpu.async_copy(buf, xs_hbm.at[idx1], sem1)
    c0.wait()
    c1.wait()


def _shared_body(x_ref, wgu_ref, wd_ref, o_ref, *, I):
    gu = lax.dot_general(x_ref[...], wgu_ref[...], (((1,), (1,)), ((), ())),
                         preferred_element_type=jnp.float32)
    gt = gu[:, :I]
    up = gu[:, I:]
    act = gt * (1.0 / (1.0 + jnp.exp(-gt))) * up
    o_ref[...] = lax.dot_general(act, wd_ref[...], (((1,), (1,)), ((), ())),
                                 preferred_element_type=jnp.float32)


def _ffn_body(te_ref, xs_ref, wgu_ref, wd_ref, o_ref, *, I):
    tev = te_ref[pl.program_id(0)]

    @pl.when(tev >= 0)
    def _():
        gu = lax.dot_general(xs_ref[...], wgu_ref[0], (((1,), (1,)), ((), ())),
                             preferred_element_type=jnp.float32)
        gt = gu[:, :I]
        up = gu[:, I:]
        act = gt * (1.0 / (1.0 + jnp.exp(-gt))) * up
        o_ref[...] = lax.dot_general(act, wd_ref[0], (((1,), (1,)), ((), ())),
                                     preferred_element_type=jnp.float32)


def _combine_body(orouted_hbm, oshared_hbm, mi_hbm, mf_hbm, out_hbm,
                  a0, a1, a2, b0, b1, s0, s1, w0b, w1b, gb, idx0, idx1,
                  sa0, sa1, sa2, sb0, sb1, ss0, ss1, so0, so1, so2,
                  *, rows, n_rounds, H):
    wid = lax.axis_index("s") * NC + lax.axis_index("c")
    nch = H // L
    chunk = rows * n_rounds
    gbase = pl.multiple_of(wid * chunk, 8)
    a_bufs, sa = (a0, a1, a2), (sa0, sa1, sa2)
    b_bufs, sb = (b0, b1), (sb0, sb1)
    s_bufs, ss = (s0, s1), (ss0, ss1)
    so = (so0, so1, so2)

    pltpu.sync_copy(mi_hbm.at[0, pl.ds(gbase, chunk)], idx0)
    pltpu.sync_copy(mi_hbm.at[1, pl.ds(gbase, chunk)], idx1)
    pltpu.sync_copy(mf_hbm.at[0, pl.ds(gbase, chunk)], w0b)
    pltpu.sync_copy(mf_hbm.at[1, pl.ds(gbase, chunk)], w1b)
    pltpu.sync_copy(mf_hbm.at[2, pl.ds(gbase, chunk)], gb)

    def issue(r):
        i0 = idx0[pl.ds(r * rows, rows)]
        i1 = idx1[pl.ds(r * rows, rows)]
        ca = pltpu.async_copy(orouted_hbm.at[i0], a_bufs[r % 3], sa[r % 3])
        cb = pltpu.async_copy(orouted_hbm.at[i1], b_bufs[r % 2], sb[r % 2])
        cs = pltpu.async_copy(
            oshared_hbm.at[pl.ds(pl.multiple_of(gbase + r * rows, 8), rows)],
            s_bufs[r % 2], ss[r % 2])
        return ca, cb, cs

    pend = {0: issue(0), 1: issue(1)}
    wouts = {}
    for r in range(n_rounds):
        for c in pend.pop(r):
            c.wait()
        a_buf, b_buf, s_buf = a_bufs[r % 3], b_bufs[r % 2], s_bufs[r % 2]

        @plsc.parallel_loop(0, rows)
        def row_body(rr, r=r, a_buf=a_buf, b_buf=b_buf, s_buf=s_buf):
            bidx = jnp.full((L,), r * rows + rr, jnp.int32)
            wv0 = plsc.load_gather(w0b, [bidx])
            wv1 = plsc.load_gather(w1b, [bidx])
            wvg = plsc.load_gather(gb, [bidx])

            @plsc.parallel_loop(0, nch, unroll=8)
            def chunk_body(ci):
                o = pl.ds(pl.multiple_of(ci * L, L), L)
                a_buf[rr, o] = (a_buf[rr, o] * wv0 + b_buf[rr, o] * wv1
                                + s_buf[rr, o] * wvg)

        wouts[r] = pltpu.async_copy(
            a_buf, out_hbm.at[pl.ds(pl.multiple_of(gbase + r * rows, 8), rows)],
            so[r % 3])
        if r + 2 < n_rounds:
            if r - 1 >= 0:
                wouts.pop(r - 1).wait()
            pend[r + 2] = issue(r + 2)
    for r in sorted(wouts):
        wouts[r].wait()


def kernel(hidden_states, gate_w, expert_gate_up_w, expert_down_w,
           shared_gate_up_w, shared_down_w, shared_expert_gate_w):
    T, H = hidden_states.shape
    E = gate_w.shape[0]
    I2 = expert_gate_up_w.shape[1]
    I = I2 // 2
    K = 2
    NRT = (T * K) // BT + E
    NRS = NRT * BT
    x = hidden_states.reshape(T, H)

    meta_i, meta_f = pl.pallas_call(
        functools.partial(_router_body, T=T, E=E, NRT=NRT),
        out_shape=(jax.ShapeDtypeStruct((8, T), jnp.int32),
                   jax.ShapeDtypeStruct((8, T), jnp.float32)),
    )(x, gate_w, shared_expert_gate_w)
    te = meta_i[2, :NRT]

    chunk = T // NW
    x_sorted = pl.kernel(
        functools.partial(_dispatch_body, chunk=chunk),
        out_type=jax.ShapeDtypeStruct((NRS, H), jnp.float32),
        mesh=plsc.VectorSubcoreMesh(core_axis_name="c", subcore_axis_name="s"),
        scratch_types=[
            pltpu.VMEM((chunk, H), jnp.float32),
            pltpu.VMEM((chunk,), jnp.int32),
            pltpu.VMEM((chunk,), jnp.int32),
            pltpu.SemaphoreType.DMA,
            pltpu.SemaphoreType.DMA,
        ],
    )(x, meta_i)

    out_shared = pl.pallas_call(
        functools.partial(_shared_body, I=I),
        grid=(T // BS,),
        in_specs=[
            pl.BlockSpec((BS, H), lambda i: (i, 0)),
            pl.BlockSpec((I2, H), lambda i: (0, 0)),
            pl.BlockSpec((H, I), lambda i: (0, 0)),
        ],
        out_specs=pl.BlockSpec((BS, H), lambda i: (i, 0)),
        out_shape=jax.ShapeDtypeStruct((T, H), jnp.float32),
    )(x, shared_gate_up_w, shared_down_w)

    out_routed = pl.pallas_call(
        functools.partial(_ffn_body, I=I),
        grid_spec=pltpu.PrefetchScalarGridSpec(
            num_scalar_prefetch=1,
            grid=(NRT,),
            in_specs=[
                pl.BlockSpec((BT, H), lambda i, s: (i, 0)),
                pl.BlockSpec((1, I2, H),
                             lambda i, s, E=E: (jnp.where(s[i] < 0, E - 1, s[i]), 0, 0)),
                pl.BlockSpec((1, H, I),
                             lambda i, s, E=E: (jnp.where(s[i] < 0, E - 1, s[i]), 0, 0)),
            ],
            out_specs=pl.BlockSpec((BT, H), lambda i, s: (i, 0)),
        ),
        out_shape=jax.ShapeDtypeStruct((NRS, H), jnp.float32),
    )(te, x_sorted, expert_gate_up_w, expert_down_w)

    rows = 16
    n_rounds = chunk // rows
    final = pl.kernel(
        functools.partial(_combine_body, rows=rows, n_rounds=n_rounds, H=H),
        out_type=jax.ShapeDtypeStruct((T, H), jnp.float32),
        mesh=plsc.VectorSubcoreMesh(core_axis_name="c", subcore_axis_name="s"),
        compiler_params=pltpu.CompilerParams(needs_layout_passes=False),
        scratch_types=(
            [pltpu.VMEM((rows, H), jnp.float32)] * 7
            + [pltpu.VMEM((chunk,), jnp.float32)] * 3
            + [pltpu.VMEM((chunk,), jnp.int32)] * 2
            + [pltpu.SemaphoreType.DMA] * 10
        ),
    )(out_routed, out_shared, meta_i, meta_f)

    return final.reshape(hidden_states.shape)


# speedup vs baseline: 3.4601x; 1.0010x over previous
import functools

import jax
import jax.numpy as jnp
from jax import lax
from jax.experimental import pallas as pl
from jax.experimental.pallas import tpu as pltpu
from jax.experimental.pallas import tpu_sc as plsc

NC, NS, L = 2, 16, 16
NW = NC * NS
BT = 512
BS = 512


def _shift_right(a, sh):
    z = jnp.zeros(a.shape[:1] + (sh,) + a.shape[2:], a.dtype)
    return jnp.concatenate([z, a[:, :-sh]], axis=1)


def _router_body(x_ref, gw_ref, sgw_ref, mi_ref, mf_ref, *, T, E, NRT):
    x = x_ref[...]
    lt = lax.dot_general(gw_ref[...], x, (((1,), (1,)), ((), ())),
                         preferred_element_type=jnp.float32)
    ls = lax.dot_general(sgw_ref[...], x, (((1,), (1,)), ((), ())),
                         preferred_element_type=jnp.float32)
    rows_e = lax.broadcasted_iota(jnp.int32, (E, T), 0)

    m = jnp.max(lt, axis=0, keepdims=True)
    p = jnp.exp(lt - m)
    p = p / jnp.sum(p, axis=0, keepdims=True)

    p1 = jnp.max(p, axis=0, keepdims=True)
    a1 = jnp.min(jnp.where(p == p1, rows_e, E), axis=0, keepdims=True)
    p_wo = jnp.where(rows_e == a1, -1.0, p)
    p2 = jnp.max(p_wo, axis=0, keepdims=True)
    a2 = jnp.min(jnp.where(p_wo == p2, rows_e, E), axis=0, keepdims=True)
    wsum = p1 + p2
    w1, w2 = p1 / wsum, p2 / wsum
    g = 1.0 / (1.0 + jnp.exp(-ls))

    m0 = (rows_e == a1)
    m1 = (rows_e == a2)
    mm = (m0 | m1).astype(jnp.int32)
    c = mm
    sh = 1
    while sh < T:
        c = c + _shift_right(c, sh)
        sh *= 2
    counts = c[:, T - 1:T]
    excl = c - mm
    rank0 = jnp.sum(jnp.where(m0, excl, 0), axis=0, keepdims=True)
    rank1 = jnp.sum(jnp.where(m1, excl, 0), axis=0, keepdims=True)

    nt = (counts + (BT - 1)) // BT
    ts = nt
    sh = 1
    while sh < E:
        z = jnp.zeros((sh, 1), jnp.int32)
        ts = ts + jnp.concatenate([z, ts[:-sh]], axis=0)
        sh *= 2
    tile_start = ts - nt
    off = tile_start * BT
    pos0 = jnp.sum(jnp.where(m0, off, 0), axis=0, keepdims=True) + rank0
    pos1 = jnp.sum(jnp.where(m1, off, 0), axis=0, keepdims=True) + rank1

    ti = lax.broadcasted_iota(jnp.int32, (E, NRT), 1)
    ts_b = jnp.broadcast_to(tile_start, (E, NRT))
    nt_b = jnp.broadcast_to(nt, (E, NRT))
    rows8 = lax.broadcasted_iota(jnp.int32, (E, NRT), 0)
    ind = ((ti >= ts_b) & (ti < ts_b + nt_b)).astype(jnp.int32)
    any_ind = jnp.sum(ind, axis=0, keepdims=True)
    te = jnp.sum(ind * rows8, axis=0, keepdims=True) - (1 - any_ind)

    mi_ref[0:1, :] = pos0
    mi_ref[1:2, :] = pos1
    mi_ref[2:3, 0:NRT] = te
    mf_ref[0:1, :] = w1
    mf_ref[1:2, :] = w2
    mf_ref[2:3, :] = g


def _dispatch_body(x_hbm, mi_hbm, xs_hbm, buf, idx0, idx1, sem0, sem1, *, chunk):
    wid = lax.axis_index("s") * NC + lax.axis_index("c")
    base = pl.multiple_of(wid * chunk, 8)
    pltpu.sync_copy(x_hbm.at[pl.ds(base, chunk)], buf)
    pltpu.sync_copy(mi_hbm.at[0, pl.ds(base, chunk)], idx0)
    pltpu.sync_copy(mi_hbm.at[1, pl.ds(base, chunk)], idx1)
    c0 = pltpu.async_copy(buf, xs_hbm.at[idx0], sem0)
    c1 = pltpu.async_copy(buf, xs_hbm.at[idx1], sem1)
    c0.wait()
    c1.wait()


def _shared_body(x_ref, wgu_ref, wd_ref, o_ref, *, I):
    gu = lax.dot_general(x_ref[...], wgu_ref[...], (((1,), (1,)), ((), ())),
                         preferred_element_type=jnp.float32)
    gt = gu[:, :I]
    up = gu[:, I:]
    act = gt * (1.0 / (1.0 + jnp.exp(-gt))) * up
    o_ref[...] = lax.dot_general(act, wd_ref[...], (((1,), (1,)), ((), ())),
                                 preferred_element_type=jnp.float32)


def _ffn_body(te_ref, xs_ref, wgu_ref, wd_ref, o_ref, *, I):
    tev = te_ref[pl.program_id(0)]

    @pl.when(tev >= 0)
    def _():
        gu = lax.dot_general(xs_ref[...], wgu_ref[0], (((1,), (1,)), ((), ())),
                             preferred_element_type=jnp.float32)
        gt = gu[:, :I]
        up = gu[:, I:]
        act = gt * (1.0 / (1.0 + jnp.exp(-gt))) * up
        o_ref[...] = lax.dot_general(act, wd_ref[0], (((1,), (1,)), ((), ())),
                                     preferred_element_type=jnp.float32)


def _combine_body(orouted_hbm, oshared_hbm, mi_hbm, mf_hbm, out_hbm,
                  a0, a1, a2, b0, b1, s0, s1, w0b, w1b, gb, idx0, idx1,
                  sa0, sa1, sa2, sb0, sb1, ss0, ss1, so0, so1, so2,
                  *, rows, n_rounds, H):
    wid = lax.axis_index("s") * NC + lax.axis_index("c")
    nch = H // L
    chunk = rows * n_rounds
    gbase = pl.multiple_of(wid * chunk, 8)
    a_bufs, sa = (a0, a1, a2), (sa0, sa1, sa2)
    b_bufs, sb = (b0, b1), (sb0, sb1)
    s_bufs, ss = (s0, s1), (ss0, ss1)
    so = (so0, so1, so2)

    pltpu.sync_copy(mi_hbm.at[0, pl.ds(gbase, chunk)], idx0)
    pltpu.sync_copy(mi_hbm.at[1, pl.ds(gbase, chunk)], idx1)
    pltpu.sync_copy(mf_hbm.at[0, pl.ds(gbase, chunk)], w0b)
    pltpu.sync_copy(mf_hbm.at[1, pl.ds(gbase, chunk)], w1b)
    pltpu.sync_copy(mf_hbm.at[2, pl.ds(gbase, chunk)], gb)

    def issue(r):
        i0 = idx0[pl.ds(r * rows, rows)]
        i1 = idx1[pl.ds(r * rows, rows)]
        ca = pltpu.async_copy(orouted_hbm.at[i0], a_bufs[r % 3], sa[r % 3])
        cb = pltpu.async_copy(orouted_hbm.at[i1], b_bufs[r % 2], sb[r % 2])
        cs = pltpu.async_copy(
            oshared_hbm.at[pl.ds(pl.multiple_of(gbase + r * rows, 8), rows)],
            s_bufs[r % 2], ss[r % 2])
        return ca, cb, cs

    pend = {0: issue(0), 1: issue(1)}
    wouts = {}
    for r in range(n_rounds):
        for c in pend.pop(r):
            c.wait()
        a_buf, b_buf, s_buf = a_bufs[r % 3], b_bufs[r % 2], s_bufs[r % 2]

        @plsc.parallel_loop(0, rows)
        def row_body(rr, r=r, a_buf=a_buf, b_buf=b_buf, s_buf=s_buf):
            bidx = jnp.full((L,), r * rows + rr, jnp.int32)
            wv0 = plsc.load_gather(w0b, [bidx])
            wv1 = plsc.load_gather(w1b, [bidx])
            wvg = plsc.load_gather(gb, [bidx])

            @plsc.parallel_loop(0, nch, unroll=8)
            def chunk_body(ci):
                o = pl.ds(pl.multiple_of(ci * L, L), L)
                a_buf[rr, o] = (a_buf[rr, o] * wv0 + b_buf[rr, o] * wv1
                                + s_buf[rr, o] * wvg)

        wouts[r] = pltpu.async_copy(
            a_buf, out_hbm.at[pl.ds(pl.multiple_of(gbase + r * rows, 8), rows)],
            so[r % 3])
        if r + 2 < n_rounds:
            if r - 1 >= 0:
                wouts.pop(r - 1).wait()
            pend[r + 2] = issue(r + 2)
    for r in sorted(wouts):
        wouts[r].wait()


def kernel(hidden_states, gate_w, expert_gate_up_w, expert_down_w,
           shared_gate_up_w, shared_down_w, shared_expert_gate_w):
    T, H = hidden_states.shape
    E = gate_w.shape[0]
    I2 = expert_gate_up_w.shape[1]
    I = I2 // 2
    K = 2
    NRT = (T * K) // BT + E
    NRS = NRT * BT
    x = hidden_states.reshape(T, H)

    meta_i, meta_f = pl.pallas_call(
        functools.partial(_router_body, T=T, E=E, NRT=NRT),
        out_shape=(jax.ShapeDtypeStruct((8, T), jnp.int32),
                   jax.ShapeDtypeStruct((8, T), jnp.float32)),
    )(x, gate_w, shared_expert_gate_w)
    te = meta_i[2, :NRT]

    chunk = T // NW
    x_sorted = pl.kernel(
        functools.partial(_dispatch_body, chunk=chunk),
        out_type=jax.ShapeDtypeStruct((NRS, H), jnp.float32),
        mesh=plsc.VectorSubcoreMesh(core_axis_name="c", subcore_axis_name="s"),
        scratch_types=[
            pltpu.VMEM((chunk, H), jnp.float32),
            pltpu.VMEM((chunk,), jnp.int32),
            pltpu.VMEM((chunk,), jnp.int32),
            pltpu.SemaphoreType.DMA,
            pltpu.SemaphoreType.DMA,
        ],
    )(x, meta_i)

    out_shared = pl.pallas_call(
        functools.partial(_shared_body, I=I),
        grid=(T // BS,),
        in_specs=[
            pl.BlockSpec((BS, H), lambda i: (i, 0)),
            pl.BlockSpec((I2, H), lambda i: (0, 0)),
            pl.BlockSpec((H, I), lambda i: (0, 0)),
        ],
        out_specs=pl.BlockSpec((BS, H), lambda i: (i, 0)),
        out_shape=jax.ShapeDtypeStruct((T, H), jnp.float32),
    )(x, shared_gate_up_w, shared_down_w)

    out_routed = pl.pallas_call(
        functools.partial(_ffn_body, I=I),
        compiler_params=pltpu.CompilerParams(
            vmem_limit_bytes=100 * 1024 * 1024),
        grid_spec=pltpu.PrefetchScalarGridSpec(
            num_scalar_prefetch=1,
            grid=(NRT,),
            in_specs=[
                pl.BlockSpec((BT, H), lambda i, s: (i, 0)),
                pl.BlockSpec((1, I2, H),
                             lambda i, s, E=E: (jnp.where(s[i] < 0, E - 1, s[i]), 0, 0)),
                pl.BlockSpec((1, H, I),
                             lambda i, s, E=E: (jnp.where(s[i] < 0, E - 1, s[i]), 0, 0)),
            ],
            out_specs=pl.BlockSpec((BT, H), lambda i, s: (i, 0)),
        ),
        out_shape=jax.ShapeDtypeStruct((NRS, H), jnp.float32),
    )(te, x_sorted, expert_gate_up_w, expert_down_w)

    rows = 16
    n_rounds = chunk // rows
    final = pl.kernel(
        functools.partial(_combine_body, rows=rows, n_rounds=n_rounds, H=H),
        out_type=jax.ShapeDtypeStruct((T, H), jnp.float32),
        mesh=plsc.VectorSubcoreMesh(core_axis_name="c", subcore_axis_name="s"),
        compiler_params=pltpu.CompilerParams(needs_layout_passes=False),
        scratch_types=(
            [pltpu.VMEM((rows, H), jnp.float32)] * 7
            + [pltpu.VMEM((chunk,), jnp.float32)] * 3
            + [pltpu.VMEM((chunk,), jnp.int32)] * 2
            + [pltpu.SemaphoreType.DMA] * 10
        ),
    )(out_routed, out_shared, meta_i, meta_f)

    return final.reshape(hidden_states.shape)


# speedup vs baseline: 3.6655x; 1.0594x over previous
import functools

import jax
import jax.numpy as jnp
from jax import lax
from jax.experimental import pallas as pl
from jax.experimental.pallas import tpu as pltpu
from jax.experimental.pallas import tpu_sc as plsc

NC, NS, L = 2, 16, 16
NW = NC * NS
BT = 512
BS = 512


def _shift_right(a, sh):
    z = jnp.zeros(a.shape[:1] + (sh,) + a.shape[2:], a.dtype)
    return jnp.concatenate([z, a[:, :-sh]], axis=1)


def _router_body(x_ref, gw_ref, sgw_ref, mi_ref, mf_ref, *, T, E, NRT):
    x = x_ref[...]
    lt = lax.dot_general(gw_ref[...], x, (((1,), (1,)), ((), ())),
                         preferred_element_type=jnp.float32)
    ls = lax.dot_general(sgw_ref[...], x, (((1,), (1,)), ((), ())),
                         preferred_element_type=jnp.float32)
    rows_e = lax.broadcasted_iota(jnp.int32, (E, T), 0)

    m = jnp.max(lt, axis=0, keepdims=True)
    p = jnp.exp(lt - m)
    p = p / jnp.sum(p, axis=0, keepdims=True)

    p1 = jnp.max(p, axis=0, keepdims=True)
    a1 = jnp.min(jnp.where(p == p1, rows_e, E), axis=0, keepdims=True)
    p_wo = jnp.where(rows_e == a1, -1.0, p)
    p2 = jnp.max(p_wo, axis=0, keepdims=True)
    a2 = jnp.min(jnp.where(p_wo == p2, rows_e, E), axis=0, keepdims=True)
    wsum = p1 + p2
    w1, w2 = p1 / wsum, p2 / wsum
    g = 1.0 / (1.0 + jnp.exp(-ls))

    m0 = (rows_e == a1)
    m1 = (rows_e == a2)
    mm = (m0 | m1).astype(jnp.int32)
    c = mm
    sh = 1
    while sh < T:
        c = c + _shift_right(c, sh)
        sh *= 2
    counts = c[:, T - 1:T]
    excl = c - mm
    rank0 = jnp.sum(jnp.where(m0, excl, 0), axis=0, keepdims=True)
    rank1 = jnp.sum(jnp.where(m1, excl, 0), axis=0, keepdims=True)

    nt = (counts + (BT - 1)) // BT
    ts = nt
    sh = 1
    while sh < E:
        z = jnp.zeros((sh, 1), jnp.int32)
        ts = ts + jnp.concatenate([z, ts[:-sh]], axis=0)
        sh *= 2
    tile_start = ts - nt
    off = tile_start * BT
    pos0 = jnp.sum(jnp.where(m0, off, 0), axis=0, keepdims=True) + rank0
    pos1 = jnp.sum(jnp.where(m1, off, 0), axis=0, keepdims=True) + rank1

    ti = lax.broadcasted_iota(jnp.int32, (E, NRT), 1)
    ts_b = jnp.broadcast_to(tile_start, (E, NRT))
    nt_b = jnp.broadcast_to(nt, (E, NRT))
    rows8 = lax.broadcasted_iota(jnp.int32, (E, NRT), 0)
    ind = ((ti >= ts_b) & (ti < ts_b + nt_b)).astype(jnp.int32)
    any_ind = jnp.sum(ind, axis=0, keepdims=True)
    te = jnp.sum(ind * rows8, axis=0, keepdims=True) - (1 - any_ind)

    mi_ref[0:1, :] = pos0
    mi_ref[1:2, :] = pos1
    mi_ref[2:3, 0:NRT] = te
    mf_ref[0:1, :] = w1
    mf_ref[1:2, :] = w2
    mf_ref[2:3, :] = g


def _dispatch_body(x_hbm, mi_hbm, xs_hbm, buf, idx0, idx1, sem0, sem1, *, chunk):
    wid = lax.axis_index("s") * NC + lax.axis_index("c")
    base = pl.multiple_of(wid * chunk, 8)
    pltpu.sync_copy(x_hbm.at[pl.ds(base, chunk)], buf)
    pltpu.sync_copy(mi_hbm.at[0, pl.ds(base, chunk)], idx0)
    pltpu.sync_copy(mi_hbm.at[1, pl.ds(base, chunk)], idx1)
    c0 = pltpu.async_copy(buf, xs_hbm.at[idx0], sem0)
    c1 = pltpu.async_copy(buf, xs_hbm.at[idx1], sem1)
    c0.wait()
    c1.wait()


def _shared_body(x_ref, wgu_ref, wd_ref, o_ref, *, I):
    gu = lax.dot_general(x_ref[...], wgu_ref[...], (((1,), (1,)), ((), ())),
                         preferred_element_type=jnp.float32)
    gt = gu[:, :I]
    up = gu[:, I:]
    act = gt * (1.0 / (1.0 + jnp.exp(-gt))) * up
    o_ref[...] = lax.dot_general(act, wd_ref[...], (((1,), (1,)), ((), ())),
                                 preferred_element_type=jnp.float32)


def _ffn_body(te_ref, xs_ref, wgu_ref, wd_ref, o_ref, *, I, H):
    tev = te_ref[pl.program_id(0)]

    @pl.when(tev >= 0)
    def _():
        gu = lax.dot_general(xs_ref[...], wgu_ref[0], (((1,), (1,)), ((), ())),
                             preferred_element_type=jnp.float32)
        gt = gu[:, :I]
        up = gu[:, I:]
        act = gt * (1.0 / (1.0 + jnp.exp(-gt))) * up
        out = lax.dot_general(act, wd_ref[0], (((1,), (1,)), ((), ())),
                              preferred_element_type=jnp.float32)
        u_lo = lax.bitcast_convert_type(out[:, :H // 2], jnp.uint32)
        u_hi = lax.bitcast_convert_type(out[:, H // 2:], jnp.uint32)
        u_lo = u_lo + 0x7FFF + ((u_lo >> 16) & 1)
        u_hi = u_hi + 0x7FFF + ((u_hi >> 16) & 1)
        o_ref[...] = lax.bitcast_convert_type(
            (u_lo >> 16) | (u_hi & jnp.uint32(0xFFFF0000)), jnp.int32)


def _combine_body(orouted_hbm, oshared_hbm, mi_hbm, mf_hbm, out_hbm,
                  a0, a1, b0, b1, s0, s1, s2, w0b, w1b, gb, idx0, idx1,
                  sa0, sa1, sb0, sb1, ss0, ss1, ss2, so0, so1, so2,
                  *, rows, n_rounds, H):
    wid = lax.axis_index("s") * NC + lax.axis_index("c")
    nch = H // L
    chunk = rows * n_rounds
    gbase = pl.multiple_of(wid * chunk, 8)
    a_bufs, sa = (a0, a1), (sa0, sa1)
    b_bufs, sb = (b0, b1), (sb0, sb1)
    s_bufs, ss = (s0, s1, s2), (ss0, ss1, ss2)
    so = (so0, so1, so2)

    pltpu.sync_copy(mi_hbm.at[0, pl.ds(gbase, chunk)], idx0)
    pltpu.sync_copy(mi_hbm.at[1, pl.ds(gbase, chunk)], idx1)
    pltpu.sync_copy(mf_hbm.at[0, pl.ds(gbase, chunk)], w0b)
    pltpu.sync_copy(mf_hbm.at[1, pl.ds(gbase, chunk)], w1b)
    pltpu.sync_copy(mf_hbm.at[2, pl.ds(gbase, chunk)], gb)

    def issue(r):
        i0 = idx0[pl.ds(r * rows, rows)]
        i1 = idx1[pl.ds(r * rows, rows)]
        ca = pltpu.async_copy(orouted_hbm.at[i0], a_bufs[r % 2], sa[r % 2])
        cb = pltpu.async_copy(orouted_hbm.at[i1], b_bufs[r % 2], sb[r % 2])
        cs = pltpu.async_copy(
            oshared_hbm.at[pl.ds(pl.multiple_of(gbase + r * rows, 8), rows)],
            s_bufs[r % 3], ss[r % 3])
        return ca, cb, cs

    pend = {0: issue(0), 1: issue(1)}
    wouts = {}
    for r in range(n_rounds):
        for c in pend.pop(r):
            c.wait()
        a_buf, b_buf, s_buf = a_bufs[r % 2], b_bufs[r % 2], s_bufs[r % 3]

        @plsc.parallel_loop(0, rows)
        def row_body(rr, r=r, a_buf=a_buf, b_buf=b_buf, s_buf=s_buf):
            bidx = jnp.full((L,), r * rows + rr, jnp.int32)
            wv0 = plsc.load_gather(w0b, [bidx])
            wv1 = plsc.load_gather(w1b, [bidx])
            wvg = plsc.load_gather(gb, [bidx])

            @plsc.parallel_loop(0, nch // 2, unroll=8)
            def chunk_body(ci):
                o = pl.ds(pl.multiple_of(ci * L, L), L)
                ohi = pl.ds(pl.multiple_of(H // 2 + ci * L, L), L)
                av = a_buf[rr, o]
                bv = b_buf[rr, o]
                a_lo = plsc.bitcast(av << 16, jnp.float32)
                a_hi = plsc.bitcast(av & jnp.int32(-65536), jnp.float32)
                b_lo = plsc.bitcast(bv << 16, jnp.float32)
                b_hi = plsc.bitcast(bv & jnp.int32(-65536), jnp.float32)
                s_buf[rr, o] = (a_lo * wv0 + b_lo * wv1
                                + s_buf[rr, o] * wvg)
                s_buf[rr, ohi] = (a_hi * wv0 + b_hi * wv1
                                  + s_buf[rr, ohi] * wvg)

        wouts[r] = pltpu.async_copy(
            s_buf, out_hbm.at[pl.ds(pl.multiple_of(gbase + r * rows, 8), rows)],
            so[r % 3])
        if r + 2 < n_rounds:
            if r - 1 >= 0:
                wouts.pop(r - 1).wait()
            pend[r + 2] = issue(r + 2)
    for r in sorted(wouts):
        wouts[r].wait()


def kernel(hidden_states, gate_w, expert_gate_up_w, expert_down_w,
           shared_gate_up_w, shared_down_w, shared_expert_gate_w):
    T, H = hidden_states.shape
    E = gate_w.shape[0]
    I2 = expert_gate_up_w.shape[1]
    I = I2 // 2
    K = 2
    NRT = (T * K) // BT + E
    NRS = NRT * BT
    x = hidden_states.reshape(T, H)

    meta_i, meta_f = pl.pallas_call(
        functools.partial(_router_body, T=T, E=E, NRT=NRT),
        out_shape=(jax.ShapeDtypeStruct((8, T), jnp.int32),
                   jax.ShapeDtypeStruct((8, T), jnp.float32)),
    )(x, gate_w, shared_expert_gate_w)
    te = meta_i[2, :NRT]

    chunk = T // NW
    x_sorted = pl.kernel(
        functools.partial(_dispatch_body, chunk=chunk),
        out_type=jax.ShapeDtypeStruct((NRS, H), jnp.float32),
        mesh=plsc.VectorSubcoreMesh(core_axis_name="c", subcore_axis_name="s"),
        scratch_types=[
            pltpu.VMEM((chunk, H), jnp.float32),
            pltpu.VMEM((chunk,), jnp.int32),
            pltpu.VMEM((chunk,), jnp.int32),
            pltpu.SemaphoreType.DMA,
            pltpu.SemaphoreType.DMA,
        ],
    )(x, meta_i)

    out_shared = pl.pallas_call(
        functools.partial(_shared_body, I=I),
        grid=(T // BS,),
        in_specs=[
            pl.BlockSpec((BS, H), lambda i: (i, 0)),
            pl.BlockSpec((I2, H), lambda i: (0, 0)),
            pl.BlockSpec((H, I), lambda i: (0, 0)),
        ],
        out_specs=pl.BlockSpec((BS, H), lambda i: (i, 0)),
        out_shape=jax.ShapeDtypeStruct((T, H), jnp.float32),
    )(x, shared_gate_up_w, shared_down_w)

    out_routed = pl.pallas_call(
        functools.partial(_ffn_body, I=I, H=H),
        compiler_params=pltpu.CompilerParams(
            vmem_limit_bytes=100 * 1024 * 1024),
        grid_spec=pltpu.PrefetchScalarGridSpec(
            num_scalar_prefetch=1,
            grid=(NRT,),
            in_specs=[
                pl.BlockSpec((BT, H), lambda i, s: (i, 0)),
                pl.BlockSpec((1, I2, H),
                             lambda i, s, E=E: (jnp.where(s[i] < 0, E - 1, s[i]), 0, 0)),
                pl.BlockSpec((1, H, I),
                             lambda i, s, E=E: (jnp.where(s[i] < 0, E - 1, s[i]), 0, 0)),
            ],
            out_specs=pl.BlockSpec((BT, H // 2), lambda i, s: (i, 0)),
        ),
        out_shape=jax.ShapeDtypeStruct((NRS, H // 2), jnp.int32),
    )(te, x_sorted, expert_gate_up_w, expert_down_w)

    rows = 16
    n_rounds = chunk // rows
    final = pl.kernel(
        functools.partial(_combine_body, rows=rows, n_rounds=n_rounds, H=H),
        out_type=jax.ShapeDtypeStruct((T, H), jnp.float32),
        mesh=plsc.VectorSubcoreMesh(core_axis_name="c", subcore_axis_name="s"),
        compiler_params=pltpu.CompilerParams(needs_layout_passes=False),
        scratch_types=(
            [pltpu.VMEM((rows, H // 2), jnp.int32)] * 4
            + [pltpu.VMEM((rows, H), jnp.float32)] * 3
            + [pltpu.VMEM((chunk,), jnp.float32)] * 3
            + [pltpu.VMEM((chunk,), jnp.int32)] * 2
            + [pltpu.SemaphoreType.DMA] * 10
        ),
    )(out_routed, out_shared, meta_i, meta_f)

    return final.reshape(hidden_states.shape)
